# baseline (speedup 1.0000x reference)
import functools

import jax
import jax.numpy as jnp
from jax import lax
from jax.experimental import pallas as pl
from jax.experimental.pallas import tpu as pltpu
from jax.experimental.pallas import tpu_sc as plsc

B, T, DIM = 16, 4096, 3
HIDDEN = 32
H2 = 2 * HIDDEN
C_DIM = 32
RESO = 128
PAD = 0.1
NB = 5
S = RESO * RESO
L = 16
GROUPS = T // L
NEG = float(jnp.finfo(jnp.float32).min)


def _dot(a, b):
    return jax.lax.dot_general(a, b, (((1,), (0,)), ((), ())),
                               preferred_element_type=jnp.float32)


def _prologue_body(pt_ref, fw_ref, fb_ref, w0_ref, b0_ref, w1_ref, b1_ref,
                   ws_ref, idx_ref, net_ref):
    pt = pt_ref[0]
    x0 = pt[0:1, :]
    x1 = pt[2:3, :]

    def norm(v):
        vn = v / (1.0 + PAD + 10e-4)
        vn = vn + 0.5
        vn = jnp.where(vn >= 1.0, 1.0 - 10e-6, vn)
        vn = jnp.where(vn < 0.0, 0.0, vn)
        return vn

    xi0 = jnp.clip((norm(x0) * RESO).astype(jnp.int32), 0, RESO - 1)
    xi1 = jnp.clip((norm(x1) * RESO).astype(jnp.int32), 0, RESO - 1)
    idx_ref[0] = xi0 + RESO * xi1

    h = _dot(fw_ref[...], pt) + fb_ref[...]
    n0 = _dot(w0_ref[...], jnp.maximum(h, 0.0)) + b0_ref[...]
    dx = _dot(w1_ref[...], jnp.maximum(n0, 0.0)) + b1_ref[...]
    net_ref[0] = _dot(ws_ref[...], h) + dx


def _res_body(net_ref, pool_ref, w0_ref, b0_ref, w1_ref, b1_ref, ws_ref,
              out_ref):
    x = jnp.concatenate([net_ref[0], pool_ref[0]], axis=0)
    n0 = _dot(w0_ref[...], jnp.maximum(x, 0.0)) + b0_ref[...]
    dx = _dot(w1_ref[...], jnp.maximum(n0, 0.0)) + b1_ref[...]
    out_ref[0] = _dot(ws_ref[...], x) + dx


def _res_final_body(net_ref, pool_ref, w0_ref, b0_ref, w1_ref, b1_ref,
                    ws_ref, fcw_ref, fcb_ref, out_ref):
    x = jnp.concatenate([net_ref[0], pool_ref[0]], axis=0)
    n0 = _dot(w0_ref[...], jnp.maximum(x, 0.0)) + b0_ref[...]
    dx = _dot(w1_ref[...], jnp.maximum(n0, 0.0)) + b1_ref[...]
    net = _dot(ws_ref[...], x) + dx
    out_ref[0] = _dot(fcw_ref[...], net) + fcb_ref[...]


def _full(shape):
    return pl.BlockSpec(shape, lambda b: (0,) * len(shape))


def _row(shape):
    return pl.BlockSpec(shape, lambda b: (b,) + (0,) * (len(shape) - 1))


_prologue_call = pl.pallas_call(
    _prologue_body,
    grid=(B,),
    in_specs=[_row((1, DIM, T)), _full((H2, DIM)), _full((H2, 1)),
              _full((HIDDEN, H2)), _full((HIDDEN, 1)),
              _full((HIDDEN, HIDDEN)), _full((HIDDEN, 1)),
              _full((HIDDEN, H2))],
    out_specs=[_row((1, 1, T)), _row((1, HIDDEN, T))],
    out_shape=[jax.ShapeDtypeStruct((B, 1, T), jnp.int32),
               jax.ShapeDtypeStruct((B, HIDDEN, T), jnp.float32)],
)

_res_call = pl.pallas_call(
    _res_body,
    grid=(B,),
    in_specs=[_row((1, HIDDEN, T)), _row((1, HIDDEN, T)),
              _full((HIDDEN, H2)), _full((HIDDEN, 1)),
              _full((HIDDEN, HIDDEN)), _full((HIDDEN, 1)),
              _full((HIDDEN, H2))],
    out_specs=_row((1, HIDDEN, T)),
    out_shape=jax.ShapeDtypeStruct((B, HIDDEN, T), jnp.float32),
)

_res_final_call = pl.pallas_call(
    _res_final_body,
    grid=(B,),
    in_specs=[_row((1, HIDDEN, T)), _row((1, HIDDEN, T)),
              _full((HIDDEN, H2)), _full((HIDDEN, 1)),
              _full((HIDDEN, HIDDEN)), _full((HIDDEN, 1)),
              _full((HIDDEN, H2)), _full((C_DIM, HIDDEN)), _full((C_DIM, 1))],
    out_specs=_row((1, C_DIM, T)),
    out_shape=jax.ShapeDtypeStruct((B, C_DIM, T), jnp.float32),
)


_MESH = plsc.VectorSubcoreMesh(core_axis_name="c", subcore_axis_name="s")
_CH_PER_W = C_DIM // 2
_SC_PARAMS = pltpu.CompilerParams(needs_layout_passes=False)


def _build_schedule(idx_v, claim, rep_v, nf_pt_v, nf_rep_v):
    lanes = lax.iota(jnp.int32, L)

    @plsc.parallel_loop(0, GROUPS, unroll=4)
    def _(g):
        idxs = idx_v[pl.ds(g * L, L)]
        plsc.store_scatter(claim, [idxs], jnp.full((L,), -1, jnp.int32))

    def build_g(g, off):
        idxs = idx_v[pl.ds(g * L, L)]
        gids = g * L + lanes
        cur = plsc.load_gather(claim, [idxs])
        free = cur == -1
        plsc.store_scatter(claim, [idxs], gids, mask=free)
        got = plsc.load_gather(claim, [idxs])
        rep = free & (got == gids)
        rep_v[pl.ds(g * L, L)] = jnp.where(rep, 1, 0)
        nf = jnp.logical_not(rep)
        nf_i = jnp.where(nf, 1, 0)
        pos = off + plsc.cumsum(nf_i) - 1
        plsc.store_scatter(nf_pt_v, [pos], gids, mask=nf)
        repgid = jnp.where(free, got, cur)
        plsc.store_scatter(nf_rep_v, [pos], repgid, mask=nf)
        return off + jnp.sum(nf_i)

    return lax.fori_loop(0, GROUPS, build_g, jnp.int32(0))


_SP = S + T
_NCHUNK = T // 128


def _chunk_scatter(src, slots, idx2d, sem):
    def start_k(k, carry):
        pltpu.async_copy(src.at[pl.ds(k * 128, 128)], slots.at[idx2d.at[k]],
                         sem)
        return carry
    lax.fori_loop(0, _NCHUNK, start_k, 0)

    def wait_k(k, carry):
        pltpu.make_async_copy(src.at[pl.ds(k * 128, 128)],
                              slots.at[idx2d.at[k]], sem).wait()
        return carry
    lax.fori_loop(0, _NCHUNK, wait_k, 0)


@functools.partial(
    pl.kernel, mesh=_MESH,
    out_type=jax.ShapeDtypeStruct((B, C_DIM, T), jnp.float32),
    compiler_params=_SC_PARAMS,
    scratch_types=[pltpu.VMEM((T,), jnp.int32),
                   pltpu.VMEM((T,), jnp.float32),
                   pltpu.VMEM((T,), jnp.float32),
                   pltpu.VMEM((S,), jnp.float32),
                   pltpu.VMEM((T,), jnp.float32),
                   pltpu.VMEM((T,), jnp.float32),
                   pltpu.VMEM((S,), jnp.int32),
                   pltpu.VMEM((T,), jnp.int32),
                   pltpu.VMEM((T,), jnp.int32),
                   pltpu.VMEM((T,), jnp.int32),
                   pltpu.SemaphoreType.DMA,
                   pltpu.SemaphoreType.DMA,
                   pltpu.SemaphoreType.DMA,
                   pltpu.SemaphoreType.DMA],
)
def _pool_call(idx_hbm, net_hbm, out_hbm, idx_v, in_a, in_b, tab, out_a,
               out_b, claim, rep_v, nf_pt_v, nf_rep_v, sem_ia, sem_ib,
               sem_oa, sem_ob):
    wid = lax.axis_index("c") * 16 + lax.axis_index("s")
    b = wid // 2
    c0 = (wid % 2) * _CH_PER_W
    pltpu.sync_copy(idx_hbm.at[b, 0], idx_v)
    in_pend = [pltpu.async_copy(net_hbm.at[b, c0], in_a, sem_ia),
               pltpu.async_copy(net_hbm.at[b, c0 + 1], in_b, sem_ib)]
    n_nf = _build_schedule(idx_v, claim, rep_v, nf_pt_v, nf_rep_v)
    n_nf_vregs = (n_nf + L - 1) // L
    lanes = lax.iota(jnp.int32, L)

    pend = [None, None]
    for ci in range(_CH_PER_W):
        p = ci % 2
        vv, ob = (in_a, out_a) if p == 0 else (in_b, out_b)
        sem_i, sem_o = (sem_ia, sem_oa) if p == 0 else (sem_ib, sem_ob)
        in_pend[p].wait()
        if pend[p] is not None:
            pend[p].wait()

        @plsc.parallel_loop(0, GROUPS, unroll=4)
        def _(g, vv=vv):
            sl = pl.ds(g * L, L)
            rep = rep_v[sl] != 0
            plsc.store_scatter(tab, [idx_v[sl]], vv[sl], mask=rep)

        def nf_pass(acc0, vv=vv):
            def nf_k(k, acc):
                valid = (k * L + lanes) < n_nf
                pts = nf_pt_v[pl.ds(k * L, L)]
                pts = jnp.where(valid, pts, 0)
                cells = plsc.load_gather(idx_v, [pts])
                vals = plsc.load_gather(vv, [pts])
                cur = plsc.load_gather(tab, [cells])
                need = valid & (vals > cur)
                plsc.store_scatter(tab, [cells], vals, mask=need)
                got = plsc.load_gather(tab, [cells])
                return acc | (valid & (vals > got))
            return lax.fori_loop(0, n_nf_vregs, nf_k, acc0)

        zero_mask = jnp.zeros((L,), jnp.bool_)
        lax.while_loop(lambda a: jnp.any(a), lambda a: nf_pass(zero_mask),
                       nf_pass(zero_mask))

        @plsc.parallel_loop(0, GROUPS, unroll=4)
        def _(g, ob=ob):
            sl = pl.ds(g * L, L)
            ob[sl] = plsc.load_gather(tab, [idx_v[sl]])

        if ci + 2 < _CH_PER_W:
            in_pend[p] = pltpu.async_copy(net_hbm.at[b, c0 + ci + 2], vv,
                                          sem_i)
        pend[p] = pltpu.async_copy(ob, out_hbm.at[b, c0 + ci], sem_o)
    pend[0].wait()
    pend[1].wait()


@functools.partial(
    pl.kernel, mesh=_MESH,
    out_type=jax.ShapeDtypeStruct((B, C_DIM, S), jnp.float32),
    compiler_params=_SC_PARAMS,
    scratch_types=[pltpu.VMEM((T,), jnp.int32),
                   pltpu.VMEM((T,), jnp.float32),
                   pltpu.VMEM((T,), jnp.float32),
                   pltpu.VMEM((T,), jnp.float32),
                   pltpu.VMEM((T,), jnp.float32),
                   pltpu.VMEM((T,), jnp.float32),
                   pltpu.VMEM((S,), jnp.float32),
                   pltpu.VMEM((S,), jnp.int32),
                   pltpu.VMEM((T,), jnp.int32),
                   pltpu.VMEM((T,), jnp.int32),
                   pltpu.VMEM((T,), jnp.int32),
                   pltpu.VMEM((_NCHUNK, 128), jnp.int32),
                   pltpu.VMEM((_NCHUNK, 128), jnp.int32),
                   pltpu.VMEM((T,), jnp.float32),
                   pltpu.VMEM((T,), jnp.int32),
                   pltpu.VMEM((T,), jnp.int32),
                   pltpu.VMEM_SHARED((16 * 2 * _SP,), jnp.float32),
                   pltpu.SemaphoreType.DMA,
                   pltpu.SemaphoreType.DMA,
                   pltpu.SemaphoreType.DMA,
                   pltpu.SemaphoreType.DMA,
                   pltpu.SemaphoreType.DMA],
)
def _mean_call(idx_hbm, c_hbm, out_hbm, idx_v, in_a, in_b, sv_a, sv_b,
               rec_pt, zeros_v, claim, rep_v, nf_pt_v, nf_rep_v, safe0,
               safe1, cnts, repof_v, nf_done, slots, sem_ia, sem_ib, sem_oa,
               sem_ob, sem_e):
    wid = lax.axis_index("c") * 16 + lax.axis_index("s")
    sid = lax.axis_index("s")
    b = wid // 2
    c0 = (wid % 2) * _CH_PER_W
    pltpu.sync_copy(idx_hbm.at[b, 0], idx_v)
    in_pend = [pltpu.async_copy(c_hbm.at[b, c0], in_a, sem_ia),
               pltpu.async_copy(c_hbm.at[b, c0 + 1], in_b, sem_ib)]
    n_nf = _build_schedule(idx_v, claim, rep_v, nf_pt_v, nf_rep_v)
    n_nf_vregs = (n_nf + L - 1) // L
    lanes = lax.iota(jnp.int32, L)
    base0 = sid * (2 * _SP)
    base1 = base0 + _SP

    @plsc.parallel_loop(0, GROUPS, unroll=4)
    def _(g):
        sl = pl.ds(g * L, L)
        row = g // 8
        col = (g % 8) * L
        cells = idx_v[sl]
        gids = g * L + lanes
        rep = rep_v[sl] != 0
        safe = jnp.where(rep, cells, S + gids)
        safe0[row, pl.ds(col, L)] = safe + base0
        safe1[row, pl.ds(col, L)] = safe + base1
        cnts[sl] = jnp.ones((L,), jnp.float32)
        repof_v[sl] = plsc.load_gather(claim, [cells])

    @plsc.parallel_loop(0, S // L, unroll=4)
    def _(g):
        zeros_v[pl.ds(g * L, L)] = jnp.zeros((L,), jnp.float32)

    zero_mask = jnp.zeros((L,), jnp.bool_)

    def _fold(dst, vals_fn):
        def reset_k(k, carry):
            nf_done[pl.ds(k * L, L)] = jnp.zeros((L,), jnp.int32)
            return carry
        lax.fori_loop(0, n_nf_vregs, reset_k, 0)

        def fold_pass(acc0):
            def fold_k(k, acc):
                sl = pl.ds(k * L, L)
                valid = (k * L + lanes) < n_nf
                pts = jnp.where(valid, nf_pt_v[sl], 0)
                reps = jnp.where(valid, nf_rep_v[sl], 0)
                a = valid & (nf_done[sl] == 0)
                cells = plsc.load_gather(idx_v, [reps])
                plsc.store_scatter(claim, [cells], lanes, mask=a)
                got = plsc.load_gather(claim, [cells])
                win = a & (got == lanes)
                vals = vals_fn(pts)
                cur = plsc.load_gather(dst, [reps])
                plsc.store_scatter(dst, [reps], cur + vals, mask=win)
                nf_done[sl] = jnp.where(win, 1, nf_done[sl])
                return acc | (a & jnp.logical_not(win))
            return lax.fori_loop(0, n_nf_vregs, fold_k, acc0)

        lax.while_loop(lambda a: jnp.any(a), lambda a: fold_pass(zero_mask),
                       fold_pass(zero_mask))

    _fold(cnts, lambda pts: jnp.ones((L,), jnp.float32))

    @plsc.parallel_loop(0, GROUPS, unroll=4)
    def _(g):
        sl = pl.ds(g * L, L)
        c = plsc.load_gather(cnts, [repof_v[sl]])
        rec_pt[sl] = 1.0 / c

    out_pend = [None, None]
    for ci in range(_CH_PER_W):
        p = ci % 2
        vv, sv = (in_a, sv_a) if p == 0 else (in_b, sv_b)
        safe = safe0 if p == 0 else safe1
        base = base0 if p == 0 else base1
        sem_i, sem_o = (sem_ia, sem_oa) if p == 0 else (sem_ib, sem_ob)
        in_pend[p].wait()

        @plsc.parallel_loop(0, GROUPS, unroll=4)
        def _(g, vv=vv, sv=sv):
            sl = pl.ds(g * L, L)
            sv[sl] = vv[sl] * rec_pt[sl]

        if ci + 2 < _CH_PER_W:
            in_pend[p] = pltpu.async_copy(c_hbm.at[b, c0 + ci + 2], vv, sem_i)

        def _sv_vals(pts, sv=sv):
            return plsc.load_gather(sv, [pts])
        _fold(sv, _sv_vals)

        if out_pend[p] is not None:
            out_pend[p].wait()
        pltpu.async_copy(zeros_v, slots.at[pl.ds(base, S)], sem_e).wait()
        _chunk_scatter(sv, slots, safe, sem_e)
        out_pend[p] = pltpu.async_copy(slots.at[pl.ds(base, S)],
                                       out_hbm.at[b, c0 + ci], sem_o)
    out_pend[0].wait()
    out_pend[1].wait()


def kernel(p, fc_pos_W, fc_pos_b, W0, b0, W1, b1, Ws, fc_c_W, fc_c_b):
    pt = jnp.transpose(p, (0, 2, 1))
    fwT = jnp.transpose(fc_pos_W)
    fbT = fc_pos_b[:, None]
    w0T = jnp.transpose(W0, (0, 2, 1))
    b0T = b0[:, :, None]
    w1T = jnp.transpose(W1, (0, 2, 1))
    b1T = b1[:, :, None]
    wsT = jnp.transpose(Ws, (0, 2, 1))
    fcwT = jnp.transpose(fc_c_W)
    fcbT = fc_c_b[:, None]

    idx, net = _prologue_call(pt, fwT, fbT, w0T[0], b0T[0], w1T[0], b1T[0],
                              wsT[0])
    for i in range(1, NB):
        pooled = _pool_call(idx, net)
        if i < NB - 1:
            net = _res_call(net, pooled, w0T[i], b0T[i], w1T[i], b1T[i],
                            wsT[i])
        else:
            c = _res_final_call(net, pooled, w0T[i], b0T[i], w1T[i], b1T[i],
                                wsT[i], fcwT, fcbT)
    plane = _mean_call(idx, c)
    return plane.reshape(B, C_DIM, RESO, RESO)


# speedup vs baseline: 1.0089x; 1.0089x over previous
import functools

import jax
import jax.numpy as jnp
from jax import lax
from jax.experimental import pallas as pl
from jax.experimental.pallas import tpu as pltpu
from jax.experimental.pallas import tpu_sc as plsc

B, T, DIM = 16, 4096, 3
HIDDEN = 32
H2 = 2 * HIDDEN
C_DIM = 32
RESO = 128
PAD = 0.1
NB = 5
S = RESO * RESO
L = 16
GROUPS = T // L
NEG = float(jnp.finfo(jnp.float32).min)


def _dot(a, b):
    return jax.lax.dot_general(a, b, (((1,), (0,)), ((), ())),
                               preferred_element_type=jnp.float32)


def _prologue_body(pt_ref, fw_ref, fb_ref, w0_ref, b0_ref, w1_ref, b1_ref,
                   ws_ref, idx_ref, net_ref):
    pt = pt_ref[0]
    x0 = pt[0:1, :]
    x1 = pt[2:3, :]

    def norm(v):
        vn = v / (1.0 + PAD + 10e-4)
        vn = vn + 0.5
        vn = jnp.where(vn >= 1.0, 1.0 - 10e-6, vn)
        vn = jnp.where(vn < 0.0, 0.0, vn)
        return vn

    xi0 = jnp.clip((norm(x0) * RESO).astype(jnp.int32), 0, RESO - 1)
    xi1 = jnp.clip((norm(x1) * RESO).astype(jnp.int32), 0, RESO - 1)
    idx_ref[0] = xi0 + RESO * xi1

    h = _dot(fw_ref[...], pt) + fb_ref[...]
    n0 = _dot(w0_ref[...], jnp.maximum(h, 0.0)) + b0_ref[...]
    dx = _dot(w1_ref[...], jnp.maximum(n0, 0.0)) + b1_ref[...]
    net_ref[0] = _dot(ws_ref[...], h) + dx


def _res_body(net_ref, pool_ref, w0_ref, b0_ref, w1_ref, b1_ref, ws_ref,
              out_ref):
    x = jnp.concatenate([net_ref[0], pool_ref[0]], axis=0)
    n0 = _dot(w0_ref[...], jnp.maximum(x, 0.0)) + b0_ref[...]
    dx = _dot(w1_ref[...], jnp.maximum(n0, 0.0)) + b1_ref[...]
    out_ref[0] = _dot(ws_ref[...], x) + dx


def _res_final_body(net_ref, pool_ref, w0_ref, b0_ref, w1_ref, b1_ref,
                    ws_ref, fcw_ref, fcb_ref, out_ref):
    x = jnp.concatenate([net_ref[0], pool_ref[0]], axis=0)
    n0 = _dot(w0_ref[...], jnp.maximum(x, 0.0)) + b0_ref[...]
    dx = _dot(w1_ref[...], jnp.maximum(n0, 0.0)) + b1_ref[...]
    net = _dot(ws_ref[...], x) + dx
    out_ref[0] = _dot(fcw_ref[...], net) + fcb_ref[...]


def _full(shape):
    return pl.BlockSpec(shape, lambda b: (0,) * len(shape))


def _row(shape):
    return pl.BlockSpec(shape, lambda b: (b,) + (0,) * (len(shape) - 1))


_prologue_call = pl.pallas_call(
    _prologue_body,
    grid=(B,),
    in_specs=[_row((1, DIM, T)), _full((H2, DIM)), _full((H2, 1)),
              _full((HIDDEN, H2)), _full((HIDDEN, 1)),
              _full((HIDDEN, HIDDEN)), _full((HIDDEN, 1)),
              _full((HIDDEN, H2))],
    out_specs=[_row((1, 1, T)), _row((1, HIDDEN, T))],
    out_shape=[jax.ShapeDtypeStruct((B, 1, T), jnp.int32),
               jax.ShapeDtypeStruct((B, HIDDEN, T), jnp.float32)],
)

_res_call = pl.pallas_call(
    _res_body,
    grid=(B,),
    in_specs=[_row((1, HIDDEN, T)), _row((1, HIDDEN, T)),
              _full((HIDDEN, H2)), _full((HIDDEN, 1)),
              _full((HIDDEN, HIDDEN)), _full((HIDDEN, 1)),
              _full((HIDDEN, H2))],
    out_specs=_row((1, HIDDEN, T)),
    out_shape=jax.ShapeDtypeStruct((B, HIDDEN, T), jnp.float32),
)

_res_final_call = pl.pallas_call(
    _res_final_body,
    grid=(B,),
    in_specs=[_row((1, HIDDEN, T)), _row((1, HIDDEN, T)),
              _full((HIDDEN, H2)), _full((HIDDEN, 1)),
              _full((HIDDEN, HIDDEN)), _full((HIDDEN, 1)),
              _full((HIDDEN, H2)), _full((C_DIM, HIDDEN)), _full((C_DIM, 1))],
    out_specs=_row((1, C_DIM, T)),
    out_shape=jax.ShapeDtypeStruct((B, C_DIM, T), jnp.float32),
)


_MESH = plsc.VectorSubcoreMesh(core_axis_name="c", subcore_axis_name="s")
_CH_PER_W = C_DIM // 2
_SC_PARAMS = pltpu.CompilerParams(needs_layout_passes=False)


def _build_schedule(idx_v, claim, rep_v, nf_pt_v, nf_rep_v):
    lanes = lax.iota(jnp.int32, L)

    @plsc.parallel_loop(0, GROUPS, unroll=4)
    def _(g):
        idxs = idx_v[pl.ds(g * L, L)]
        plsc.store_scatter(claim, [idxs], jnp.full((L,), -1, jnp.int32))

    def build_g(g, off):
        idxs = idx_v[pl.ds(g * L, L)]
        gids = g * L + lanes
        cur = plsc.load_gather(claim, [idxs])
        free = cur == -1
        plsc.store_scatter(claim, [idxs], gids, mask=free)
        got = plsc.load_gather(claim, [idxs])
        rep = free & (got == gids)
        rep_v[pl.ds(g * L, L)] = jnp.where(rep, 1, 0)
        nf = jnp.logical_not(rep)
        nf_i = jnp.where(nf, 1, 0)
        pos = off + plsc.cumsum(nf_i) - 1
        plsc.store_scatter(nf_pt_v, [pos], gids, mask=nf)
        repgid = jnp.where(free, got, cur)
        plsc.store_scatter(nf_rep_v, [pos], repgid, mask=nf)
        return off + jnp.sum(nf_i)

    return lax.fori_loop(0, GROUPS, build_g, jnp.int32(0))


_SP = S + T
_NCHUNK = T // 128


def _chunk_scatter(src, slots, idx2d, sem):
    def start_k(k, carry):
        pltpu.async_copy(src.at[pl.ds(k * 128, 128)], slots.at[idx2d.at[k]],
                         sem)
        return carry
    lax.fori_loop(0, _NCHUNK, start_k, 0)

    def wait_k(k, carry):
        pltpu.make_async_copy(src.at[pl.ds(k * 128, 128)],
                              slots.at[idx2d.at[k]], sem).wait()
        return carry
    lax.fori_loop(0, _NCHUNK, wait_k, 0)


@functools.partial(
    pl.kernel, mesh=_MESH,
    out_type=jax.ShapeDtypeStruct((B, C_DIM, T), jnp.float32),
    compiler_params=_SC_PARAMS,
    scratch_types=[pltpu.VMEM((T,), jnp.int32),
                   pltpu.VMEM((T,), jnp.float32),
                   pltpu.VMEM((T,), jnp.float32),
                   pltpu.VMEM((S,), jnp.float32),
                   pltpu.VMEM((T,), jnp.float32),
                   pltpu.VMEM((T,), jnp.float32),
                   pltpu.VMEM((S,), jnp.int32),
                   pltpu.VMEM((T,), jnp.int32),
                   pltpu.VMEM((T,), jnp.int32),
                   pltpu.VMEM((T,), jnp.int32),
                   pltpu.SemaphoreType.DMA,
                   pltpu.SemaphoreType.DMA,
                   pltpu.SemaphoreType.DMA,
                   pltpu.SemaphoreType.DMA],
)
def _pool_call(idx_hbm, net_hbm, out_hbm, idx_v, in_a, in_b, tab, out_a,
               out_b, claim, rep_v, nf_pt_v, nf_rep_v, sem_ia, sem_ib,
               sem_oa, sem_ob):
    wid = lax.axis_index("c") * 16 + lax.axis_index("s")
    b = wid // 2
    c0 = (wid % 2) * _CH_PER_W
    pltpu.sync_copy(idx_hbm.at[b, 0], idx_v)
    in_pend = [pltpu.async_copy(net_hbm.at[b, c0], in_a, sem_ia),
               pltpu.async_copy(net_hbm.at[b, c0 + 1], in_b, sem_ib)]
    n_nf = _build_schedule(idx_v, claim, rep_v, nf_pt_v, nf_rep_v)
    n_nf_vregs = (n_nf + L - 1) // L
    lanes = lax.iota(jnp.int32, L)

    pend = [None, None]
    for ci in range(_CH_PER_W):
        p = ci % 2
        vv, ob = (in_a, out_a) if p == 0 else (in_b, out_b)
        sem_i, sem_o = (sem_ia, sem_oa) if p == 0 else (sem_ib, sem_ob)
        in_pend[p].wait()
        if pend[p] is not None:
            pend[p].wait()

        @plsc.parallel_loop(0, GROUPS, unroll=4)
        def _(g, vv=vv):
            sl = pl.ds(g * L, L)
            rep = rep_v[sl] != 0
            plsc.store_scatter(tab, [idx_v[sl]], vv[sl], mask=rep)

        def nf_pass(acc0, vv=vv):
            def nf_k(k, acc):
                valid = (k * L + lanes) < n_nf
                pts = nf_pt_v[pl.ds(k * L, L)]
                pts = jnp.where(valid, pts, 0)
                cells = plsc.load_gather(idx_v, [pts])
                vals = plsc.load_gather(vv, [pts])
                cur = plsc.load_gather(tab, [cells])
                need = valid & (vals > cur)
                plsc.store_scatter(tab, [cells], vals, mask=need)
                got = plsc.load_gather(tab, [cells])
                return acc | (valid & (vals > got))
            return lax.fori_loop(0, n_nf_vregs, nf_k, acc0)

        zero_mask = jnp.zeros((L,), jnp.bool_)
        lax.while_loop(lambda more: more,
                       lambda more: jnp.any(nf_pass(zero_mask)),
                       jnp.bool_(True))

        @plsc.parallel_loop(0, GROUPS, unroll=4)
        def _(g, ob=ob):
            sl = pl.ds(g * L, L)
            ob[sl] = plsc.load_gather(tab, [idx_v[sl]])

        if ci + 2 < _CH_PER_W:
            in_pend[p] = pltpu.async_copy(net_hbm.at[b, c0 + ci + 2], vv,
                                          sem_i)
        pend[p] = pltpu.async_copy(ob, out_hbm.at[b, c0 + ci], sem_o)
    pend[0].wait()
    pend[1].wait()


@functools.partial(
    pl.kernel, mesh=_MESH,
    out_type=jax.ShapeDtypeStruct((B, C_DIM, S), jnp.float32),
    compiler_params=_SC_PARAMS,
    scratch_types=[pltpu.VMEM((T,), jnp.int32),
                   pltpu.VMEM((T,), jnp.float32),
                   pltpu.VMEM((T,), jnp.float32),
                   pltpu.VMEM((T,), jnp.float32),
                   pltpu.VMEM((T,), jnp.float32),
                   pltpu.VMEM((T,), jnp.float32),
                   pltpu.VMEM((S,), jnp.float32),
                   pltpu.VMEM((S,), jnp.int32),
                   pltpu.VMEM((T,), jnp.int32),
                   pltpu.VMEM((T,), jnp.int32),
                   pltpu.VMEM((T,), jnp.int32),
                   pltpu.VMEM((_NCHUNK, 128), jnp.int32),
                   pltpu.VMEM((_NCHUNK, 128), jnp.int32),
                   pltpu.VMEM((T,), jnp.float32),
                   pltpu.VMEM((T,), jnp.int32),
                   pltpu.VMEM((T,), jnp.int32),
                   pltpu.VMEM_SHARED((16 * 2 * _SP,), jnp.float32),
                   pltpu.SemaphoreType.DMA,
                   pltpu.SemaphoreType.DMA,
                   pltpu.SemaphoreType.DMA,
                   pltpu.SemaphoreType.DMA,
                   pltpu.SemaphoreType.DMA],
)
def _mean_call(idx_hbm, c_hbm, out_hbm, idx_v, in_a, in_b, sv_a, sv_b,
               rec_pt, zeros_v, claim, rep_v, nf_pt_v, nf_rep_v, safe0,
               safe1, cnts, repof_v, nf_done, slots, sem_ia, sem_ib, sem_oa,
               sem_ob, sem_e):
    wid = lax.axis_index("c") * 16 + lax.axis_index("s")
    sid = lax.axis_index("s")
    b = wid // 2
    c0 = (wid % 2) * _CH_PER_W
    pltpu.sync_copy(idx_hbm.at[b, 0], idx_v)
    in_pend = [pltpu.async_copy(c_hbm.at[b, c0], in_a, sem_ia),
               pltpu.async_copy(c_hbm.at[b, c0 + 1], in_b, sem_ib)]
    n_nf = _build_schedule(idx_v, claim, rep_v, nf_pt_v, nf_rep_v)
    n_nf_vregs = (n_nf + L - 1) // L
    lanes = lax.iota(jnp.int32, L)
    base0 = sid * (2 * _SP)
    base1 = base0 + _SP

    @plsc.parallel_loop(0, GROUPS, unroll=4)
    def _(g):
        sl = pl.ds(g * L, L)
        row = g // 8
        col = (g % 8) * L
        cells = idx_v[sl]
        gids = g * L + lanes
        rep = rep_v[sl] != 0
        safe = jnp.where(rep, cells, S + gids)
        safe0[row, pl.ds(col, L)] = safe + base0
        safe1[row, pl.ds(col, L)] = safe + base1
        cnts[sl] = jnp.ones((L,), jnp.float32)
        repof_v[sl] = plsc.load_gather(claim, [cells])

    @plsc.parallel_loop(0, S // L, unroll=4)
    def _(g):
        zeros_v[pl.ds(g * L, L)] = jnp.zeros((L,), jnp.float32)

    zero_mask = jnp.zeros((L,), jnp.bool_)

    def _fold(dst, vals_fn):
        def reset_k(k, carry):
            nf_done[pl.ds(k * L, L)] = jnp.zeros((L,), jnp.int32)
            return carry
        lax.fori_loop(0, n_nf_vregs, reset_k, 0)

        def fold_pass(acc0):
            def fold_k(k, acc):
                sl = pl.ds(k * L, L)
                valid = (k * L + lanes) < n_nf
                pts = jnp.where(valid, nf_pt_v[sl], 0)
                reps = jnp.where(valid, nf_rep_v[sl], 0)
                a = valid & (nf_done[sl] == 0)
                cells = plsc.load_gather(idx_v, [reps])
                plsc.store_scatter(claim, [cells], lanes, mask=a)
                got = plsc.load_gather(claim, [cells])
                win = a & (got == lanes)
                vals = vals_fn(pts)
                cur = plsc.load_gather(dst, [reps])
                plsc.store_scatter(dst, [reps], cur + vals, mask=win)
                nf_done[sl] = jnp.where(win, 1, nf_done[sl])
                return acc | (a & jnp.logical_not(win))
            return lax.fori_loop(0, n_nf_vregs, fold_k, acc0)

        lax.while_loop(lambda more: more,
                       lambda more: jnp.any(fold_pass(zero_mask)),
                       jnp.bool_(True))

    _fold(cnts, lambda pts: jnp.ones((L,), jnp.float32))

    @plsc.parallel_loop(0, GROUPS, unroll=4)
    def _(g):
        sl = pl.ds(g * L, L)
        c = plsc.load_gather(cnts, [repof_v[sl]])
        rec_pt[sl] = 1.0 / c

    out_pend = [None, None]
    for ci in range(_CH_PER_W):
        p = ci % 2
        vv, sv = (in_a, sv_a) if p == 0 else (in_b, sv_b)
        safe = safe0 if p == 0 else safe1
        base = base0 if p == 0 else base1
        sem_i, sem_o = (sem_ia, sem_oa) if p == 0 else (sem_ib, sem_ob)
        in_pend[p].wait()

        @plsc.parallel_loop(0, GROUPS, unroll=4)
        def _(g, vv=vv, sv=sv):
            sl = pl.ds(g * L, L)
            sv[sl] = vv[sl] * rec_pt[sl]

        if ci + 2 < _CH_PER_W:
            in_pend[p] = pltpu.async_copy(c_hbm.at[b, c0 + ci + 2], vv, sem_i)

        def _sv_vals(pts, sv=sv):
            return plsc.load_gather(sv, [pts])
        _fold(sv, _sv_vals)

        if out_pend[p] is not None:
            out_pend[p].wait()
        pltpu.async_copy(zeros_v, slots.at[pl.ds(base, S)], sem_e).wait()
        _chunk_scatter(sv, slots, safe, sem_e)
        out_pend[p] = pltpu.async_copy(slots.at[pl.ds(base, S)],
                                       out_hbm.at[b, c0 + ci], sem_o)
    out_pend[0].wait()
    out_pend[1].wait()


def kernel(p, fc_pos_W, fc_pos_b, W0, b0, W1, b1, Ws, fc_c_W, fc_c_b):
    pt = jnp.transpose(p, (0, 2, 1))
    fwT = jnp.transpose(fc_pos_W)
    fbT = fc_pos_b[:, None]
    w0T = jnp.transpose(W0, (0, 2, 1))
    b0T = b0[:, :, None]
    w1T = jnp.transpose(W1, (0, 2, 1))
    b1T = b1[:, :, None]
    wsT = jnp.transpose(Ws, (0, 2, 1))
    fcwT = jnp.transpose(fc_c_W)
    fcbT = fc_c_b[:, None]

    idx, net = _prologue_call(pt, fwT, fbT, w0T[0], b0T[0], w1T[0], b1T[0],
                              wsT[0])
    for i in range(1, NB):
        pooled = _pool_call(idx, net)
        if i < NB - 1:
            net = _res_call(net, pooled, w0T[i], b0T[i], w1T[i], b1T[i],
                            wsT[i])
        else:
            c = _res_final_call(net, pooled, w0T[i], b0T[i], w1T[i], b1T[i],
                                wsT[i], fcwT, fcbT)
    plane = _mean_call(idx, c)
    return plane.reshape(B, C_DIM, RESO, RESO)


# speedup vs baseline: 1.0209x; 1.0119x over previous
import functools

import jax
import jax.numpy as jnp
from jax import lax
from jax.experimental import pallas as pl
from jax.experimental.pallas import tpu as pltpu
from jax.experimental.pallas import tpu_sc as plsc

B, T, DIM = 16, 4096, 3
HIDDEN = 32
H2 = 2 * HIDDEN
C_DIM = 32
RESO = 128
PAD = 0.1
NB = 5
S = RESO * RESO
L = 16
GROUPS = T // L
NEG = float(jnp.finfo(jnp.float32).min)


def _dot(a, b):
    return jax.lax.dot_general(a, b, (((1,), (0,)), ((), ())),
                               preferred_element_type=jnp.float32)


def _prologue_body(pt_ref, fw_ref, fb_ref, w0_ref, b0_ref, w1_ref, b1_ref,
                   ws_ref, idx_ref, net_ref):
    pt = pt_ref[0]
    x0 = pt[0:1, :]
    x1 = pt[2:3, :]

    def norm(v):
        vn = v / (1.0 + PAD + 10e-4)
        vn = vn + 0.5
        vn = jnp.where(vn >= 1.0, 1.0 - 10e-6, vn)
        vn = jnp.where(vn < 0.0, 0.0, vn)
        return vn

    xi0 = jnp.clip((norm(x0) * RESO).astype(jnp.int32), 0, RESO - 1)
    xi1 = jnp.clip((norm(x1) * RESO).astype(jnp.int32), 0, RESO - 1)
    idx_ref[0] = xi0 + RESO * xi1

    h = _dot(fw_ref[...], pt) + fb_ref[...]
    n0 = _dot(w0_ref[...], jnp.maximum(h, 0.0)) + b0_ref[...]
    dx = _dot(w1_ref[...], jnp.maximum(n0, 0.0)) + b1_ref[...]
    net_ref[0] = _dot(ws_ref[...], h) + dx


def _res_body(net_ref, pool_ref, w0_ref, b0_ref, w1_ref, b1_ref, ws_ref,
              out_ref):
    x = jnp.concatenate([net_ref[0], pool_ref[0]], axis=0)
    n0 = _dot(w0_ref[...], jnp.maximum(x, 0.0)) + b0_ref[...]
    dx = _dot(w1_ref[...], jnp.maximum(n0, 0.0)) + b1_ref[...]
    out_ref[0] = _dot(ws_ref[...], x) + dx


def _res_final_body(net_ref, pool_ref, w0_ref, b0_ref, w1_ref, b1_ref,
                    ws_ref, fcw_ref, fcb_ref, out_ref):
    x = jnp.concatenate([net_ref[0], pool_ref[0]], axis=0)
    n0 = _dot(w0_ref[...], jnp.maximum(x, 0.0)) + b0_ref[...]
    dx = _dot(w1_ref[...], jnp.maximum(n0, 0.0)) + b1_ref[...]
    net = _dot(ws_ref[...], x) + dx
    out_ref[0] = _dot(fcw_ref[...], net) + fcb_ref[...]


def _full(shape):
    return pl.BlockSpec(shape, lambda b: (0,) * len(shape))


def _row(shape):
    return pl.BlockSpec(shape, lambda b: (b,) + (0,) * (len(shape) - 1))


_prologue_call = pl.pallas_call(
    _prologue_body,
    grid=(B,),
    in_specs=[_row((1, DIM, T)), _full((H2, DIM)), _full((H2, 1)),
              _full((HIDDEN, H2)), _full((HIDDEN, 1)),
              _full((HIDDEN, HIDDEN)), _full((HIDDEN, 1)),
              _full((HIDDEN, H2))],
    out_specs=[_row((1, 1, T)), _row((1, HIDDEN, T))],
    out_shape=[jax.ShapeDtypeStruct((B, 1, T), jnp.int32),
               jax.ShapeDtypeStruct((B, HIDDEN, T), jnp.float32)],
)

_res_call = pl.pallas_call(
    _res_body,
    grid=(B,),
    in_specs=[_row((1, HIDDEN, T)), _row((1, HIDDEN, T)),
              _full((HIDDEN, H2)), _full((HIDDEN, 1)),
              _full((HIDDEN, HIDDEN)), _full((HIDDEN, 1)),
              _full((HIDDEN, H2))],
    out_specs=_row((1, HIDDEN, T)),
    out_shape=jax.ShapeDtypeStruct((B, HIDDEN, T), jnp.float32),
)

_res_final_call = pl.pallas_call(
    _res_final_body,
    grid=(B,),
    in_specs=[_row((1, HIDDEN, T)), _row((1, HIDDEN, T)),
              _full((HIDDEN, H2)), _full((HIDDEN, 1)),
              _full((HIDDEN, HIDDEN)), _full((HIDDEN, 1)),
              _full((HIDDEN, H2)), _full((C_DIM, HIDDEN)), _full((C_DIM, 1))],
    out_specs=_row((1, C_DIM, T)),
    out_shape=jax.ShapeDtypeStruct((B, C_DIM, T), jnp.float32),
)


_MESH = plsc.VectorSubcoreMesh(core_axis_name="c", subcore_axis_name="s")
_CH_PER_W = C_DIM // 2
_SC_PARAMS = pltpu.CompilerParams(needs_layout_passes=False)


def _build_schedule(idx_v, claim, rep_v, nf_pt_v, nf_rep_v):
    lanes = lax.iota(jnp.int32, L)

    @plsc.parallel_loop(0, GROUPS, unroll=4)
    def _(g):
        idxs = idx_v[pl.ds(g * L, L)]
        plsc.store_scatter(claim, [idxs], jnp.full((L,), -1, jnp.int32))

    def build_g(g, off):
        idxs = idx_v[pl.ds(g * L, L)]
        gids = g * L + lanes
        cur = plsc.load_gather(claim, [idxs])
        free = cur == -1
        plsc.store_scatter(claim, [idxs], gids, mask=free)
        got = plsc.load_gather(claim, [idxs])
        rep = free & (got == gids)
        rep_v[pl.ds(g * L, L)] = jnp.where(rep, 1, 0)
        nf = jnp.logical_not(rep)
        nf_i = jnp.where(nf, 1, 0)
        pos = off + plsc.cumsum(nf_i) - 1
        plsc.store_scatter(nf_pt_v, [pos], gids, mask=nf)
        repgid = jnp.where(free, got, cur)
        plsc.store_scatter(nf_rep_v, [pos], repgid, mask=nf)
        return off + jnp.sum(nf_i)

    return lax.fori_loop(0, GROUPS, build_g, jnp.int32(0))


@functools.partial(
    pl.kernel, mesh=_MESH,
    out_type=jax.ShapeDtypeStruct((B, C_DIM, T), jnp.float32),
    compiler_params=_SC_PARAMS,
    scratch_types=[pltpu.VMEM((T,), jnp.int32),
                   pltpu.VMEM((T,), jnp.float32),
                   pltpu.VMEM((T,), jnp.float32),
                   pltpu.VMEM((S,), jnp.float32),
                   pltpu.VMEM((T,), jnp.float32),
                   pltpu.VMEM((T,), jnp.float32),
                   pltpu.VMEM((S,), jnp.int32),
                   pltpu.VMEM((T,), jnp.int32),
                   pltpu.VMEM((T,), jnp.int32),
                   pltpu.VMEM((T,), jnp.int32),
                   pltpu.SemaphoreType.DMA,
                   pltpu.SemaphoreType.DMA,
                   pltpu.SemaphoreType.DMA,
                   pltpu.SemaphoreType.DMA],
)
def _pool_call(idx_hbm, net_hbm, out_hbm, idx_v, in_a, in_b, tab, out_a,
               out_b, claim, rep_v, nf_pt_v, nf_rep_v, sem_ia, sem_ib,
               sem_oa, sem_ob):
    wid = lax.axis_index("c") * 16 + lax.axis_index("s")
    b = wid // 2
    c0 = (wid % 2) * _CH_PER_W
    pltpu.sync_copy(idx_hbm.at[b, 0], idx_v)
    in_pend = [pltpu.async_copy(net_hbm.at[b, c0], in_a, sem_ia),
               pltpu.async_copy(net_hbm.at[b, c0 + 1], in_b, sem_ib)]
    n_nf = _build_schedule(idx_v, claim, rep_v, nf_pt_v, nf_rep_v)
    n_nf_vregs = (n_nf + L - 1) // L
    lanes = lax.iota(jnp.int32, L)

    pend = [None, None]
    for ci in range(_CH_PER_W):
        p = ci % 2
        vv, ob = (in_a, out_a) if p == 0 else (in_b, out_b)
        sem_i, sem_o = (sem_ia, sem_oa) if p == 0 else (sem_ib, sem_ob)
        in_pend[p].wait()
        if pend[p] is not None:
            pend[p].wait()

        @plsc.parallel_loop(0, GROUPS, unroll=4)
        def _(g, vv=vv):
            sl = pl.ds(g * L, L)
            rep = rep_v[sl] != 0
            plsc.store_scatter(tab, [idx_v[sl]], vv[sl], mask=rep)

        def nf_pass(acc0, vv=vv):
            def nf_k(k, acc):
                valid = (k * L + lanes) < n_nf
                pts = nf_pt_v[pl.ds(k * L, L)]
                pts = jnp.where(valid, pts, 0)
                cells = plsc.load_gather(idx_v, [pts])
                vals = plsc.load_gather(vv, [pts])
                cur = plsc.load_gather(tab, [cells])
                need = valid & (vals > cur)
                plsc.store_scatter(tab, [cells], vals, mask=need)
                got = plsc.load_gather(tab, [cells])
                return acc | (valid & (vals > got))
            return lax.fori_loop(0, n_nf_vregs, nf_k, acc0)

        zero_mask = jnp.zeros((L,), jnp.bool_)
        lax.while_loop(lambda more: more,
                       lambda more: jnp.any(nf_pass(zero_mask)),
                       jnp.bool_(True))

        @plsc.parallel_loop(0, GROUPS, unroll=4)
        def _(g, ob=ob):
            sl = pl.ds(g * L, L)
            ob[sl] = plsc.load_gather(tab, [idx_v[sl]])

        if ci + 2 < _CH_PER_W:
            in_pend[p] = pltpu.async_copy(net_hbm.at[b, c0 + ci + 2], vv,
                                          sem_i)
        pend[p] = pltpu.async_copy(ob, out_hbm.at[b, c0 + ci], sem_o)
    pend[0].wait()
    pend[1].wait()


@functools.partial(
    pl.kernel, mesh=_MESH,
    out_type=jax.ShapeDtypeStruct((B, C_DIM, S), jnp.float32),
    compiler_params=_SC_PARAMS,
    scratch_types=[pltpu.VMEM((T,), jnp.int32),
                   pltpu.VMEM((T,), jnp.float32),
                   pltpu.VMEM((T,), jnp.float32),
                   pltpu.VMEM((T,), jnp.float32),
                   pltpu.VMEM((T,), jnp.float32),
                   pltpu.VMEM((T,), jnp.float32),
                   pltpu.VMEM((S,), jnp.float32),
                   pltpu.VMEM((S,), jnp.float32),
                   pltpu.VMEM((S,), jnp.int32),
                   pltpu.VMEM((T,), jnp.int32),
                   pltpu.VMEM((T,), jnp.int32),
                   pltpu.VMEM((T,), jnp.int32),
                   pltpu.VMEM((T,), jnp.float32),
                   pltpu.VMEM((T,), jnp.int32),
                   pltpu.VMEM((T,), jnp.int32),
                   pltpu.SemaphoreType.DMA,
                   pltpu.SemaphoreType.DMA,
                   pltpu.SemaphoreType.DMA,
                   pltpu.SemaphoreType.DMA],
)
def _mean_call(idx_hbm, c_hbm, out_hbm, idx_v, in_a, in_b, sv_a, sv_b,
               rec_pt, tab_a, tab_b, claim, rep_v, nf_pt_v, nf_rep_v,
               cnts, repof_v, nf_done, sem_ia, sem_ib, sem_oa, sem_ob):
    wid = lax.axis_index("c") * 16 + lax.axis_index("s")
    b = wid // 2
    c0 = (wid % 2) * _CH_PER_W
    pltpu.sync_copy(idx_hbm.at[b, 0], idx_v)
    in_pend = [pltpu.async_copy(c_hbm.at[b, c0], in_a, sem_ia),
               pltpu.async_copy(c_hbm.at[b, c0 + 1], in_b, sem_ib)]
    n_nf = _build_schedule(idx_v, claim, rep_v, nf_pt_v, nf_rep_v)
    n_nf_vregs = (n_nf + L - 1) // L
    lanes = lax.iota(jnp.int32, L)

    @plsc.parallel_loop(0, GROUPS, unroll=4)
    def _(g):
        sl = pl.ds(g * L, L)
        cells = idx_v[sl]
        cnts[sl] = jnp.ones((L,), jnp.float32)
        repof_v[sl] = plsc.load_gather(claim, [cells])

    @plsc.parallel_loop(0, S // L, unroll=4)
    def _(g):
        tab_a[pl.ds(g * L, L)] = jnp.zeros((L,), jnp.float32)

    @plsc.parallel_loop(0, S // L, unroll=4)
    def _(g):
        tab_b[pl.ds(g * L, L)] = jnp.zeros((L,), jnp.float32)

    zero_mask = jnp.zeros((L,), jnp.bool_)

    def _fold(dst, vals_fn):
        def reset_k(k, carry):
            nf_done[pl.ds(k * L, L)] = jnp.zeros((L,), jnp.int32)
            return carry
        lax.fori_loop(0, n_nf_vregs, reset_k, 0)

        def fold_pass(acc0):
            def fold_k(k, acc):
                sl = pl.ds(k * L, L)
                valid = (k * L + lanes) < n_nf
                pts = jnp.where(valid, nf_pt_v[sl], 0)
                reps = jnp.where(valid, nf_rep_v[sl], 0)
                a = valid & (nf_done[sl] == 0)
                cells = plsc.load_gather(idx_v, [reps])
                plsc.store_scatter(claim, [cells], lanes, mask=a)
                got = plsc.load_gather(claim, [cells])
                win = a & (got == lanes)
                vals = vals_fn(pts)
                cur = plsc.load_gather(dst, [reps])
                plsc.store_scatter(dst, [reps], cur + vals, mask=win)
                nf_done[sl] = jnp.where(win, 1, nf_done[sl])
                return acc | (a & jnp.logical_not(win))
            return lax.fori_loop(0, n_nf_vregs, fold_k, acc0)

        lax.while_loop(lambda more: more,
                       lambda more: jnp.any(fold_pass(zero_mask)),
                       jnp.bool_(True))

    _fold(cnts, lambda pts: jnp.ones((L,), jnp.float32))

    @plsc.parallel_loop(0, GROUPS, unroll=4)
    def _(g):
        sl = pl.ds(g * L, L)
        c = plsc.load_gather(cnts, [repof_v[sl]])
        rec_pt[sl] = 1.0 / c

    out_pend = [None, None]
    for ci in range(_CH_PER_W):
        p = ci % 2
        vv, sv, tb = (in_a, sv_a, tab_a) if p == 0 else (in_b, sv_b, tab_b)
        sem_i, sem_o = (sem_ia, sem_oa) if p == 0 else (sem_ib, sem_ob)
        in_pend[p].wait()

        @plsc.parallel_loop(0, GROUPS, unroll=4)
        def _(g, vv=vv, sv=sv):
            sl = pl.ds(g * L, L)
            sv[sl] = vv[sl] * rec_pt[sl]

        if ci + 2 < _CH_PER_W:
            in_pend[p] = pltpu.async_copy(c_hbm.at[b, c0 + ci + 2], vv, sem_i)

        def _sv_vals(pts, sv=sv):
            return plsc.load_gather(sv, [pts])
        _fold(sv, _sv_vals)

        if out_pend[p] is not None:
            out_pend[p].wait()

        @plsc.parallel_loop(0, GROUPS, unroll=4)
        def _(g, sv=sv, tb=tb):
            sl = pl.ds(g * L, L)
            rep = rep_v[sl] != 0
            plsc.store_scatter(tb, [idx_v[sl]], sv[sl], mask=rep)

        out_pend[p] = pltpu.async_copy(tb, out_hbm.at[b, c0 + ci], sem_o)
    out_pend[0].wait()
    out_pend[1].wait()


def kernel(p, fc_pos_W, fc_pos_b, W0, b0, W1, b1, Ws, fc_c_W, fc_c_b):
    pt = jnp.transpose(p, (0, 2, 1))
    fwT = jnp.transpose(fc_pos_W)
    fbT = fc_pos_b[:, None]
    w0T = jnp.transpose(W0, (0, 2, 1))
    b0T = b0[:, :, None]
    w1T = jnp.transpose(W1, (0, 2, 1))
    b1T = b1[:, :, None]
    wsT = jnp.transpose(Ws, (0, 2, 1))
    fcwT = jnp.transpose(fc_c_W)
    fcbT = fc_c_b[:, None]

    idx, net = _prologue_call(pt, fwT, fbT, w0T[0], b0T[0], w1T[0], b1T[0],
                              wsT[0])
    for i in range(1, NB):
        pooled = _pool_call(idx, net)
        if i < NB - 1:
            net = _res_call(net, pooled, w0T[i], b0T[i], w1T[i], b1T[i],
                            wsT[i])
        else:
            c = _res_final_call(net, pooled, w0T[i], b0T[i], w1T[i], b1T[i],
                                wsT[i], fcwT, fcbT)
    plane = _mean_call(idx, c)
    return plane.reshape(B, C_DIM, RESO, RESO)


# speedup vs baseline: 1.6862x; 1.6517x over previous
import functools

import jax
import jax.numpy as jnp
from jax import lax
from jax.experimental import pallas as pl
from jax.experimental.pallas import tpu as pltpu
from jax.experimental.pallas import tpu_sc as plsc

B, T, DIM = 16, 4096, 3
HIDDEN = 32
H2 = 2 * HIDDEN
C_DIM = 32
RESO = 128
PAD = 0.1
NB = 5
S = RESO * RESO
L = 16
GROUPS = T // L
NEG = float(jnp.finfo(jnp.float32).min)


def _dot(a, b):
    return jax.lax.dot_general(a, b, (((1,), (0,)), ((), ())),
                               preferred_element_type=jnp.float32)


def _prologue_body(pt_ref, fw_ref, fb_ref, w0_ref, b0_ref, w1_ref, b1_ref,
                   ws_ref, idx_ref, net_ref):
    pt = pt_ref[0]
    x0 = pt[0:1, :]
    x1 = pt[2:3, :]

    def norm(v):
        vn = v / (1.0 + PAD + 10e-4)
        vn = vn + 0.5
        vn = jnp.where(vn >= 1.0, 1.0 - 10e-6, vn)
        vn = jnp.where(vn < 0.0, 0.0, vn)
        return vn

    xi0 = jnp.clip((norm(x0) * RESO).astype(jnp.int32), 0, RESO - 1)
    xi1 = jnp.clip((norm(x1) * RESO).astype(jnp.int32), 0, RESO - 1)
    idx_ref[0] = xi0 + RESO * xi1

    h = _dot(fw_ref[...], pt) + fb_ref[...]
    n0 = _dot(w0_ref[...], jnp.maximum(h, 0.0)) + b0_ref[...]
    dx = _dot(w1_ref[...], jnp.maximum(n0, 0.0)) + b1_ref[...]
    net_ref[0] = _dot(ws_ref[...], h) + dx


def _res_body(net_ref, pool_ref, w0_ref, b0_ref, w1_ref, b1_ref, ws_ref,
              out_ref):
    x = jnp.concatenate([net_ref[0], pool_ref[0]], axis=0)
    n0 = _dot(w0_ref[...], jnp.maximum(x, 0.0)) + b0_ref[...]
    dx = _dot(w1_ref[...], jnp.maximum(n0, 0.0)) + b1_ref[...]
    out_ref[0] = _dot(ws_ref[...], x) + dx


def _res_final_body(net_ref, pool_ref, w0_ref, b0_ref, w1_ref, b1_ref,
                    ws_ref, fcw_ref, fcb_ref, out_ref):
    x = jnp.concatenate([net_ref[0], pool_ref[0]], axis=0)
    n0 = _dot(w0_ref[...], jnp.maximum(x, 0.0)) + b0_ref[...]
    dx = _dot(w1_ref[...], jnp.maximum(n0, 0.0)) + b1_ref[...]
    net = _dot(ws_ref[...], x) + dx
    out_ref[0] = _dot(fcw_ref[...], net) + fcb_ref[...]


def _full(shape):
    return pl.BlockSpec(shape, lambda b: (0,) * len(shape))


def _row(shape):
    return pl.BlockSpec(shape, lambda b: (b,) + (0,) * (len(shape) - 1))


_prologue_call = pl.pallas_call(
    _prologue_body,
    grid=(B,),
    in_specs=[_row((1, DIM, T)), _full((H2, DIM)), _full((H2, 1)),
              _full((HIDDEN, H2)), _full((HIDDEN, 1)),
              _full((HIDDEN, HIDDEN)), _full((HIDDEN, 1)),
              _full((HIDDEN, H2))],
    out_specs=[_row((1, 1, T)), _row((1, HIDDEN, T))],
    out_shape=[jax.ShapeDtypeStruct((B, 1, T), jnp.int32),
               jax.ShapeDtypeStruct((B, HIDDEN, T), jnp.float32)],
)

_res_call = pl.pallas_call(
    _res_body,
    grid=(B,),
    in_specs=[_row((1, HIDDEN, T)), _row((1, HIDDEN, T)),
              _full((HIDDEN, H2)), _full((HIDDEN, 1)),
              _full((HIDDEN, HIDDEN)), _full((HIDDEN, 1)),
              _full((HIDDEN, H2))],
    out_specs=_row((1, HIDDEN, T)),
    out_shape=jax.ShapeDtypeStruct((B, HIDDEN, T), jnp.float32),
)

_res_final_call = pl.pallas_call(
    _res_final_body,
    grid=(B,),
    in_specs=[_row((1, HIDDEN, T)), _row((1, HIDDEN, T)),
              _full((HIDDEN, H2)), _full((HIDDEN, 1)),
              _full((HIDDEN, HIDDEN)), _full((HIDDEN, 1)),
              _full((HIDDEN, H2)), _full((C_DIM, HIDDEN)), _full((C_DIM, 1))],
    out_specs=_row((1, C_DIM, T)),
    out_shape=jax.ShapeDtypeStruct((B, C_DIM, T), jnp.float32),
)


_MESH = plsc.VectorSubcoreMesh(core_axis_name="c", subcore_axis_name="s")
_CH_PER_W = C_DIM // 2
_SC_PARAMS = pltpu.CompilerParams(needs_layout_passes=False)


def _build_schedule(idx_v, claim, rep_v, nf_pt_v, nf_rep_v):
    lanes = lax.iota(jnp.int32, L)

    @plsc.parallel_loop(0, GROUPS, unroll=4)
    def _(g):
        idxs = idx_v[pl.ds(g * L, L)]
        plsc.store_scatter(claim, [idxs], jnp.full((L,), -1, jnp.int32))

    def build_g(g, off):
        idxs = idx_v[pl.ds(g * L, L)]
        gids = g * L + lanes
        cur = plsc.load_gather(claim, [idxs])
        free = cur == -1
        plsc.store_scatter(claim, [idxs], gids, mask=free)
        got = plsc.load_gather(claim, [idxs])
        rep = free & (got == gids)
        rep_v[pl.ds(g * L, L)] = jnp.where(rep, 1, 0)
        nf = jnp.logical_not(rep)
        nf_i = jnp.where(nf, 1, 0)
        pos = off + plsc.cumsum(nf_i) - 1
        plsc.store_scatter(nf_pt_v, [pos], gids, mask=nf)
        repgid = jnp.where(free, got, cur)
        plsc.store_scatter(nf_rep_v, [pos], repgid, mask=nf)
        return off + jnp.sum(nf_i)

    return lax.fori_loop(0, GROUPS, build_g, jnp.int32(0))


@functools.partial(
    pl.kernel, mesh=_MESH,
    out_type=jax.ShapeDtypeStruct((B, C_DIM, T), jnp.float32),
    compiler_params=_SC_PARAMS,
    scratch_types=[pltpu.VMEM((T,), jnp.int32),
                   pltpu.VMEM((T,), jnp.float32),
                   pltpu.VMEM((T,), jnp.float32),
                   pltpu.VMEM((S,), jnp.float32),
                   pltpu.VMEM((T,), jnp.float32),
                   pltpu.VMEM((T,), jnp.float32),
                   pltpu.VMEM((S,), jnp.int32),
                   pltpu.VMEM((T,), jnp.int32),
                   pltpu.VMEM((T,), jnp.int32),
                   pltpu.VMEM((T,), jnp.int32),
                   pltpu.SemaphoreType.DMA,
                   pltpu.SemaphoreType.DMA,
                   pltpu.SemaphoreType.DMA,
                   pltpu.SemaphoreType.DMA],
)
def _pool_call(idx_hbm, net_hbm, out_hbm, idx_v, in_a, in_b, tab, out_a,
               out_b, claim, rep_v, nf_pt_v, nf_rep_v, sem_ia, sem_ib,
               sem_oa, sem_ob):
    wid = lax.axis_index("c") * 16 + lax.axis_index("s")
    b = wid // 2
    c0 = (wid % 2) * _CH_PER_W
    pltpu.sync_copy(idx_hbm.at[b, 0], idx_v)
    in_pend = [pltpu.async_copy(net_hbm.at[b, c0], in_a, sem_ia),
               pltpu.async_copy(net_hbm.at[b, c0 + 1], in_b, sem_ib)]
    n_nf = _build_schedule(idx_v, claim, rep_v, nf_pt_v, nf_rep_v)
    n_nf_vregs = (n_nf + L - 1) // L
    lanes = lax.iota(jnp.int32, L)

    pend = [None, None]
    for ci in range(_CH_PER_W):
        p = ci % 2
        vv, ob = (in_a, out_a) if p == 0 else (in_b, out_b)
        sem_i, sem_o = (sem_ia, sem_oa) if p == 0 else (sem_ib, sem_ob)
        in_pend[p].wait()
        if pend[p] is not None:
            pend[p].wait()

        @plsc.parallel_loop(0, GROUPS, unroll=4)
        def _(g, vv=vv):
            sl = pl.ds(g * L, L)
            rep = rep_v[sl] != 0
            plsc.store_scatter(tab, [idx_v[sl]], vv[sl], mask=rep)

        def nf_pass(acc0, vv=vv):
            def nf_k(k, acc):
                valid = (k * L + lanes) < n_nf
                pts = nf_pt_v[pl.ds(k * L, L)]
                pts = jnp.where(valid, pts, 0)
                cells = plsc.load_gather(idx_v, [pts])
                vals = plsc.load_gather(vv, [pts])
                cur = plsc.load_gather(tab, [cells])
                need = valid & (vals > cur)
                plsc.store_scatter(tab, [cells], vals, mask=need)
                got = plsc.load_gather(tab, [cells])
                return acc | (valid & (vals > got))
            return lax.fori_loop(0, n_nf_vregs, nf_k, acc0)

        zero_mask = jnp.zeros((L,), jnp.bool_)
        lax.while_loop(lambda more: more,
                       lambda more: jnp.any(nf_pass(zero_mask)),
                       jnp.bool_(True))

        @plsc.parallel_loop(0, GROUPS, unroll=4)
        def _(g, ob=ob):
            sl = pl.ds(g * L, L)
            ob[sl] = plsc.load_gather(tab, [idx_v[sl]])

        if ci + 2 < _CH_PER_W:
            in_pend[p] = pltpu.async_copy(net_hbm.at[b, c0 + ci + 2], vv,
                                          sem_i)
        pend[p] = pltpu.async_copy(ob, out_hbm.at[b, c0 + ci], sem_o)
    pend[0].wait()
    pend[1].wait()


@functools.partial(
    pl.kernel, mesh=_MESH,
    out_type=jax.ShapeDtypeStruct((B, C_DIM, S), jnp.float32),
    compiler_params=_SC_PARAMS,
    scratch_types=[pltpu.VMEM((T,), jnp.int32),
                   pltpu.VMEM((T,), jnp.float32),
                   pltpu.VMEM((T,), jnp.float32),
                   pltpu.VMEM((T,), jnp.float32),
                   pltpu.VMEM((T,), jnp.float32),
                   pltpu.VMEM((T,), jnp.float32),
                   pltpu.VMEM((S,), jnp.float32),
                   pltpu.VMEM((S,), jnp.float32),
                   pltpu.VMEM((S,), jnp.int32),
                   pltpu.VMEM((T,), jnp.int32),
                   pltpu.VMEM((T,), jnp.int32),
                   pltpu.VMEM((T,), jnp.int32),
                   pltpu.VMEM((T,), jnp.float32),
                   pltpu.VMEM((T,), jnp.int32),
                   pltpu.VMEM((L,), jnp.int32),
                   pltpu.VMEM((L,), jnp.float32),
                   pltpu.SemaphoreType.DMA,
                   pltpu.SemaphoreType.DMA,
                   pltpu.SemaphoreType.DMA,
                   pltpu.SemaphoreType.DMA],
)
def _mean_call(idx_hbm, c_hbm, out_hbm, idx_v, in_a, in_b, sv_a, sv_b,
               rec_pt, tab_a, tab_b, claim, rep_v, nf_pt_v, nf_rep_v,
               cnts, repof_v, shk, shv, sem_ia, sem_ib, sem_oa, sem_ob):
    wid = lax.axis_index("c") * 16 + lax.axis_index("s")
    b = wid // 2
    c0 = (wid % 2) * _CH_PER_W
    pltpu.sync_copy(idx_hbm.at[b, 0], idx_v)
    in_pend = [pltpu.async_copy(c_hbm.at[b, c0], in_a, sem_ia),
               pltpu.async_copy(c_hbm.at[b, c0 + 1], in_b, sem_ib)]
    n_nf = _build_schedule(idx_v, claim, rep_v, nf_pt_v, nf_rep_v)
    n_nf_vregs = (n_nf + L - 1) // L
    lanes = lax.iota(jnp.int32, L)

    @plsc.parallel_loop(0, GROUPS, unroll=4)
    def _(g):
        sl = pl.ds(g * L, L)
        cells = idx_v[sl]
        cnts[sl] = jnp.ones((L,), jnp.float32)
        repof_v[sl] = plsc.load_gather(claim, [cells])

    @plsc.parallel_loop(0, S // L, unroll=4)
    def _(g):
        tab_a[pl.ds(g * L, L)] = jnp.zeros((L,), jnp.float32)

    @plsc.parallel_loop(0, S // L, unroll=4)
    def _(g):
        tab_b[pl.ds(g * L, L)] = jnp.zeros((L,), jnp.float32)

    zero_mask = jnp.zeros((L,), jnp.bool_)

    _SENT = jnp.int32(0x7FFFFFFF)

    def _fold(dst, vals_fn):
        def fold_k(k, carry):
            sl = pl.ds(k * L, L)
            valid = (k * L + lanes) < n_nf
            pts = jnp.where(valid, nf_pt_v[sl], 0)
            reps = jnp.where(valid, nf_rep_v[sl], _SENT)
            vals = jnp.where(valid, vals_fn(pts), 0.0)
            kk, vv = plsc.sort_key_val(reps, vals)
            shk[...] = kk
            for s in (1, 2, 4, 8):
                idxs = jnp.maximum(lanes - s, 0)
                ks = plsc.load_gather(shk, [idxs])
                shv[...] = vv
                vs = plsc.load_gather(shv, [idxs])
                vv = jnp.where((ks == kk) & (lanes >= s), vv + vs, vv)
            kn = plsc.load_gather(shk, [jnp.minimum(lanes + 1, L - 1)])
            do = ((kn != kk) | (lanes == L - 1)) & (kk != _SENT)
            tgt = jnp.where(do, kk, 0)
            cur = plsc.load_gather(dst, [tgt])
            plsc.store_scatter(dst, [tgt], cur + vv, mask=do)
            return carry
        lax.fori_loop(0, n_nf_vregs, fold_k, 0)

    _fold(cnts, lambda pts: jnp.ones((L,), jnp.float32))

    @plsc.parallel_loop(0, GROUPS, unroll=4)
    def _(g):
        sl = pl.ds(g * L, L)
        c = plsc.load_gather(cnts, [repof_v[sl]])
        rec_pt[sl] = 1.0 / c

    out_pend = [None, None]
    for ci in range(_CH_PER_W):
        p = ci % 2
        vv, sv, tb = (in_a, sv_a, tab_a) if p == 0 else (in_b, sv_b, tab_b)
        sem_i, sem_o = (sem_ia, sem_oa) if p == 0 else (sem_ib, sem_ob)
        in_pend[p].wait()

        @plsc.parallel_loop(0, GROUPS, unroll=4)
        def _(g, vv=vv, sv=sv):
            sl = pl.ds(g * L, L)
            sv[sl] = vv[sl] * rec_pt[sl]

        if ci + 2 < _CH_PER_W:
            in_pend[p] = pltpu.async_copy(c_hbm.at[b, c0 + ci + 2], vv, sem_i)

        def _sv_vals(pts, sv=sv):
            return plsc.load_gather(sv, [pts])
        _fold(sv, _sv_vals)

        if out_pend[p] is not None:
            out_pend[p].wait()

        @plsc.parallel_loop(0, GROUPS, unroll=4)
        def _(g, sv=sv, tb=tb):
            sl = pl.ds(g * L, L)
            rep = rep_v[sl] != 0
            plsc.store_scatter(tb, [idx_v[sl]], sv[sl], mask=rep)

        out_pend[p] = pltpu.async_copy(tb, out_hbm.at[b, c0 + ci], sem_o)
    out_pend[0].wait()
    out_pend[1].wait()


def kernel(p, fc_pos_W, fc_pos_b, W0, b0, W1, b1, Ws, fc_c_W, fc_c_b):
    pt = jnp.transpose(p, (0, 2, 1))
    fwT = jnp.transpose(fc_pos_W)
    fbT = fc_pos_b[:, None]
    w0T = jnp.transpose(W0, (0, 2, 1))
    b0T = b0[:, :, None]
    w1T = jnp.transpose(W1, (0, 2, 1))
    b1T = b1[:, :, None]
    wsT = jnp.transpose(Ws, (0, 2, 1))
    fcwT = jnp.transpose(fc_c_W)
    fcbT = fc_c_b[:, None]

    idx, net = _prologue_call(pt, fwT, fbT, w0T[0], b0T[0], w1T[0], b1T[0],
                              wsT[0])
    for i in range(1, NB):
        pooled = _pool_call(idx, net)
        if i < NB - 1:
            net = _res_call(net, pooled, w0T[i], b0T[i], w1T[i], b1T[i],
                            wsT[i])
        else:
            c = _res_final_call(net, pooled, w0T[i], b0T[i], w1T[i], b1T[i],
                                wsT[i], fcwT, fcbT)
    plane = _mean_call(idx, c)
    return plane.reshape(B, C_DIM, RESO, RESO)


# speedup vs baseline: 1.7715x; 1.0506x over previous
import functools

import jax
import jax.numpy as jnp
from jax import lax
from jax.experimental import pallas as pl
from jax.experimental.pallas import tpu as pltpu
from jax.experimental.pallas import tpu_sc as plsc

B, T, DIM = 16, 4096, 3
HIDDEN = 32
H2 = 2 * HIDDEN
C_DIM = 32
RESO = 128
PAD = 0.1
NB = 5
S = RESO * RESO
L = 16
GROUPS = T // L
NEG = float(jnp.finfo(jnp.float32).min)


def _dot(a, b):
    return jax.lax.dot_general(a, b, (((1,), (0,)), ((), ())),
                               preferred_element_type=jnp.float32)


def _prologue_body(pt_ref, fw_ref, fb_ref, w0_ref, b0_ref, w1_ref, b1_ref,
                   ws_ref, idx_ref, net_ref):
    pt = pt_ref[0]
    x0 = pt[0:1, :]
    x1 = pt[2:3, :]

    def norm(v):
        vn = v / (1.0 + PAD + 10e-4)
        vn = vn + 0.5
        vn = jnp.where(vn >= 1.0, 1.0 - 10e-6, vn)
        vn = jnp.where(vn < 0.0, 0.0, vn)
        return vn

    xi0 = jnp.clip((norm(x0) * RESO).astype(jnp.int32), 0, RESO - 1)
    xi1 = jnp.clip((norm(x1) * RESO).astype(jnp.int32), 0, RESO - 1)
    idx_ref[0] = xi0 + RESO * xi1

    h = _dot(fw_ref[...], pt) + fb_ref[...]
    n0 = _dot(w0_ref[...], jnp.maximum(h, 0.0)) + b0_ref[...]
    dx = _dot(w1_ref[...], jnp.maximum(n0, 0.0)) + b1_ref[...]
    net_ref[0] = _dot(ws_ref[...], h) + dx


def _res_body(net_ref, pool_ref, w0_ref, b0_ref, w1_ref, b1_ref, ws_ref,
              out_ref):
    x = jnp.concatenate([net_ref[0], pool_ref[0]], axis=0)
    n0 = _dot(w0_ref[...], jnp.maximum(x, 0.0)) + b0_ref[...]
    dx = _dot(w1_ref[...], jnp.maximum(n0, 0.0)) + b1_ref[...]
    out_ref[0] = _dot(ws_ref[...], x) + dx


def _res_final_body(net_ref, pool_ref, w0_ref, b0_ref, w1_ref, b1_ref,
                    ws_ref, fcw_ref, fcb_ref, out_ref):
    x = jnp.concatenate([net_ref[0], pool_ref[0]], axis=0)
    n0 = _dot(w0_ref[...], jnp.maximum(x, 0.0)) + b0_ref[...]
    dx = _dot(w1_ref[...], jnp.maximum(n0, 0.0)) + b1_ref[...]
    net = _dot(ws_ref[...], x) + dx
    out_ref[0] = _dot(fcw_ref[...], net) + fcb_ref[...]


def _full(shape):
    return pl.BlockSpec(shape, lambda b: (0,) * len(shape))


def _row(shape):
    return pl.BlockSpec(shape, lambda b: (b,) + (0,) * (len(shape) - 1))


_prologue_call = pl.pallas_call(
    _prologue_body,
    grid=(B,),
    in_specs=[_row((1, DIM, T)), _full((H2, DIM)), _full((H2, 1)),
              _full((HIDDEN, H2)), _full((HIDDEN, 1)),
              _full((HIDDEN, HIDDEN)), _full((HIDDEN, 1)),
              _full((HIDDEN, H2))],
    out_specs=[_row((1, 1, T)), _row((1, HIDDEN, T))],
    out_shape=[jax.ShapeDtypeStruct((B, 1, T), jnp.int32),
               jax.ShapeDtypeStruct((B, HIDDEN, T), jnp.float32)],
)

_res_call = pl.pallas_call(
    _res_body,
    grid=(B,),
    in_specs=[_row((1, HIDDEN, T)), _row((1, HIDDEN, T)),
              _full((HIDDEN, H2)), _full((HIDDEN, 1)),
              _full((HIDDEN, HIDDEN)), _full((HIDDEN, 1)),
              _full((HIDDEN, H2))],
    out_specs=_row((1, HIDDEN, T)),
    out_shape=jax.ShapeDtypeStruct((B, HIDDEN, T), jnp.float32),
)

_res_final_call = pl.pallas_call(
    _res_final_body,
    grid=(B,),
    in_specs=[_row((1, HIDDEN, T)), _row((1, HIDDEN, T)),
              _full((HIDDEN, H2)), _full((HIDDEN, 1)),
              _full((HIDDEN, HIDDEN)), _full((HIDDEN, 1)),
              _full((HIDDEN, H2)), _full((C_DIM, HIDDEN)), _full((C_DIM, 1))],
    out_specs=_row((1, C_DIM, T)),
    out_shape=jax.ShapeDtypeStruct((B, C_DIM, T), jnp.float32),
)


_MESH = plsc.VectorSubcoreMesh(core_axis_name="c", subcore_axis_name="s")
_CH_PER_W = C_DIM // 2
_SC_PARAMS = pltpu.CompilerParams(needs_layout_passes=False)


def _build_schedule(idx_v, claim, rep_v, nf_pt_v, nf_rep_v):
    lanes = lax.iota(jnp.int32, L)

    @plsc.parallel_loop(0, GROUPS, unroll=4)
    def _(g):
        idxs = idx_v[pl.ds(g * L, L)]
        plsc.store_scatter(claim, [idxs], jnp.full((L,), -1, jnp.int32))

    def build_g(g, off):
        idxs = idx_v[pl.ds(g * L, L)]
        gids = g * L + lanes
        cur = plsc.load_gather(claim, [idxs])
        free = cur == -1
        plsc.store_scatter(claim, [idxs], gids, mask=free)
        got = plsc.load_gather(claim, [idxs])
        rep = free & (got == gids)
        rep_v[pl.ds(g * L, L)] = jnp.where(rep, 1, 0)
        nf = jnp.logical_not(rep)
        nf_i = jnp.where(nf, 1, 0)
        pos = off + plsc.cumsum(nf_i) - 1
        plsc.store_scatter(nf_pt_v, [pos], gids, mask=nf)
        repgid = jnp.where(free, got, cur)
        plsc.store_scatter(nf_rep_v, [pos], repgid, mask=nf)
        return off + jnp.sum(nf_i)

    return lax.fori_loop(0, GROUPS, build_g, jnp.int32(0))


@functools.partial(
    pl.kernel, mesh=_MESH,
    out_type=jax.ShapeDtypeStruct((B, C_DIM, T), jnp.float32),
    compiler_params=_SC_PARAMS,
    scratch_types=[pltpu.VMEM((T,), jnp.int32),
                   pltpu.VMEM((T,), jnp.float32),
                   pltpu.VMEM((T,), jnp.float32),
                   pltpu.VMEM((S,), jnp.float32),
                   pltpu.VMEM((T,), jnp.float32),
                   pltpu.VMEM((T,), jnp.float32),
                   pltpu.VMEM((S,), jnp.int32),
                   pltpu.VMEM((T,), jnp.int32),
                   pltpu.VMEM((T,), jnp.int32),
                   pltpu.VMEM((T,), jnp.int32),
                   pltpu.VMEM((L,), jnp.int32),
                   pltpu.VMEM((L,), jnp.float32),
                   pltpu.SemaphoreType.DMA,
                   pltpu.SemaphoreType.DMA,
                   pltpu.SemaphoreType.DMA,
                   pltpu.SemaphoreType.DMA],
)
def _pool_call(idx_hbm, net_hbm, out_hbm, idx_v, in_a, in_b, tab, out_a,
               out_b, claim, rep_v, nf_pt_v, nf_rep_v, shk, shv, sem_ia,
               sem_ib, sem_oa, sem_ob):
    wid = lax.axis_index("c") * 16 + lax.axis_index("s")
    b = wid // 2
    c0 = (wid % 2) * _CH_PER_W
    pltpu.sync_copy(idx_hbm.at[b, 0], idx_v)
    in_pend = [pltpu.async_copy(net_hbm.at[b, c0], in_a, sem_ia),
               pltpu.async_copy(net_hbm.at[b, c0 + 1], in_b, sem_ib)]
    n_nf = _build_schedule(idx_v, claim, rep_v, nf_pt_v, nf_rep_v)
    n_nf_vregs = (n_nf + L - 1) // L
    lanes = lax.iota(jnp.int32, L)

    pend = [None, None]
    for ci in range(_CH_PER_W):
        p = ci % 2
        vv, ob = (in_a, out_a) if p == 0 else (in_b, out_b)
        sem_i, sem_o = (sem_ia, sem_oa) if p == 0 else (sem_ib, sem_ob)
        in_pend[p].wait()
        if pend[p] is not None:
            pend[p].wait()

        @plsc.parallel_loop(0, GROUPS, unroll=4)
        def _(g, vv=vv):
            sl = pl.ds(g * L, L)
            rep = rep_v[sl] != 0
            plsc.store_scatter(tab, [idx_v[sl]], vv[sl], mask=rep)

        def nf_k(k, carry2, vv=vv):
            valid = (k * L + lanes) < n_nf
            pts = jnp.where(valid, nf_pt_v[pl.ds(k * L, L)], 0)
            cells = plsc.load_gather(idx_v, [pts])
            cells = jnp.where(valid, cells, jnp.int32(0x7FFFFFFF))
            vals = plsc.load_gather(vv, [pts])
            vals = jnp.where(valid, vals, NEG)
            kk, vx = plsc.sort_key_val(cells, vals)
            shk[...] = kk
            for s in (1, 2, 4, 8):
                idxs = jnp.maximum(lanes - s, 0)
                ks = plsc.load_gather(shk, [idxs])
                shv[...] = vx
                vs = plsc.load_gather(shv, [idxs])
                vx = jnp.where((ks == kk) & (lanes >= s),
                               jnp.maximum(vx, vs), vx)
            kn = plsc.load_gather(shk, [jnp.minimum(lanes + 1, L - 1)])
            do = ((kn != kk) | (lanes == L - 1)) & (kk != 0x7FFFFFFF)
            tgt = jnp.where(do, kk, 0)
            cur = plsc.load_gather(tab, [tgt])
            plsc.store_scatter(tab, [tgt], jnp.maximum(cur, vx), mask=do)
            return carry2
        lax.fori_loop(0, n_nf_vregs, nf_k, 0)

        @plsc.parallel_loop(0, GROUPS, unroll=4)
        def _(g, ob=ob):
            sl = pl.ds(g * L, L)
            ob[sl] = plsc.load_gather(tab, [idx_v[sl]])

        if ci + 2 < _CH_PER_W:
            in_pend[p] = pltpu.async_copy(net_hbm.at[b, c0 + ci + 2], vv,
                                          sem_i)
        pend[p] = pltpu.async_copy(ob, out_hbm.at[b, c0 + ci], sem_o)
    pend[0].wait()
    pend[1].wait()


@functools.partial(
    pl.kernel, mesh=_MESH,
    out_type=jax.ShapeDtypeStruct((B, C_DIM, S), jnp.float32),
    compiler_params=_SC_PARAMS,
    scratch_types=[pltpu.VMEM((T,), jnp.int32),
                   pltpu.VMEM((T,), jnp.float32),
                   pltpu.VMEM((T,), jnp.float32),
                   pltpu.VMEM((T,), jnp.float32),
                   pltpu.VMEM((T,), jnp.float32),
                   pltpu.VMEM((T,), jnp.float32),
                   pltpu.VMEM((S,), jnp.float32),
                   pltpu.VMEM((S,), jnp.float32),
                   pltpu.VMEM((S,), jnp.int32),
                   pltpu.VMEM((T,), jnp.int32),
                   pltpu.VMEM((T,), jnp.int32),
                   pltpu.VMEM((T,), jnp.int32),
                   pltpu.VMEM((T,), jnp.float32),
                   pltpu.VMEM((T,), jnp.int32),
                   pltpu.VMEM((L,), jnp.int32),
                   pltpu.VMEM((L,), jnp.float32),
                   pltpu.SemaphoreType.DMA,
                   pltpu.SemaphoreType.DMA,
                   pltpu.SemaphoreType.DMA,
                   pltpu.SemaphoreType.DMA],
)
def _mean_call(idx_hbm, c_hbm, out_hbm, idx_v, in_a, in_b, sv_a, sv_b,
               rec_pt, tab_a, tab_b, claim, rep_v, nf_pt_v, nf_rep_v,
               cnts, repof_v, shk, shv, sem_ia, sem_ib, sem_oa, sem_ob):
    wid = lax.axis_index("c") * 16 + lax.axis_index("s")
    b = wid // 2
    c0 = (wid % 2) * _CH_PER_W
    pltpu.sync_copy(idx_hbm.at[b, 0], idx_v)
    in_pend = [pltpu.async_copy(c_hbm.at[b, c0], in_a, sem_ia),
               pltpu.async_copy(c_hbm.at[b, c0 + 1], in_b, sem_ib)]
    n_nf = _build_schedule(idx_v, claim, rep_v, nf_pt_v, nf_rep_v)
    n_nf_vregs = (n_nf + L - 1) // L
    lanes = lax.iota(jnp.int32, L)

    @plsc.parallel_loop(0, GROUPS, unroll=4)
    def _(g):
        sl = pl.ds(g * L, L)
        cells = idx_v[sl]
        cnts[sl] = jnp.ones((L,), jnp.float32)
        repof_v[sl] = plsc.load_gather(claim, [cells])

    @plsc.parallel_loop(0, S // L, unroll=4)
    def _(g):
        tab_a[pl.ds(g * L, L)] = jnp.zeros((L,), jnp.float32)

    @plsc.parallel_loop(0, S // L, unroll=4)
    def _(g):
        tab_b[pl.ds(g * L, L)] = jnp.zeros((L,), jnp.float32)

    zero_mask = jnp.zeros((L,), jnp.bool_)

    _SENT = jnp.int32(0x7FFFFFFF)

    def _fold(dst, vals_fn):
        def fold_k(k, carry):
            sl = pl.ds(k * L, L)
            valid = (k * L + lanes) < n_nf
            pts = jnp.where(valid, nf_pt_v[sl], 0)
            reps = jnp.where(valid, nf_rep_v[sl], _SENT)
            vals = jnp.where(valid, vals_fn(pts), 0.0)
            kk, vv = plsc.sort_key_val(reps, vals)
            shk[...] = kk
            for s in (1, 2, 4, 8):
                idxs = jnp.maximum(lanes - s, 0)
                ks = plsc.load_gather(shk, [idxs])
                shv[...] = vv
                vs = plsc.load_gather(shv, [idxs])
                vv = jnp.where((ks == kk) & (lanes >= s), vv + vs, vv)
            kn = plsc.load_gather(shk, [jnp.minimum(lanes + 1, L - 1)])
            do = ((kn != kk) | (lanes == L - 1)) & (kk != _SENT)
            tgt = jnp.where(do, kk, 0)
            cur = plsc.load_gather(dst, [tgt])
            plsc.store_scatter(dst, [tgt], cur + vv, mask=do)
            return carry
        lax.fori_loop(0, n_nf_vregs, fold_k, 0)

    _fold(cnts, lambda pts: jnp.ones((L,), jnp.float32))

    @plsc.parallel_loop(0, GROUPS, unroll=4)
    def _(g):
        sl = pl.ds(g * L, L)
        c = plsc.load_gather(cnts, [repof_v[sl]])
        rec_pt[sl] = 1.0 / c

    out_pend = [None, None]
    for ci in range(_CH_PER_W):
        p = ci % 2
        vv, sv, tb = (in_a, sv_a, tab_a) if p == 0 else (in_b, sv_b, tab_b)
        sem_i, sem_o = (sem_ia, sem_oa) if p == 0 else (sem_ib, sem_ob)
        in_pend[p].wait()

        @plsc.parallel_loop(0, GROUPS, unroll=4)
        def _(g, vv=vv, sv=sv):
            sl = pl.ds(g * L, L)
            sv[sl] = vv[sl] * rec_pt[sl]

        if ci + 2 < _CH_PER_W:
            in_pend[p] = pltpu.async_copy(c_hbm.at[b, c0 + ci + 2], vv, sem_i)

        def _sv_vals(pts, sv=sv):
            return plsc.load_gather(sv, [pts])
        _fold(sv, _sv_vals)

        if out_pend[p] is not None:
            out_pend[p].wait()

        @plsc.parallel_loop(0, GROUPS, unroll=4)
        def _(g, sv=sv, tb=tb):
            sl = pl.ds(g * L, L)
            rep = rep_v[sl] != 0
            plsc.store_scatter(tb, [idx_v[sl]], sv[sl], mask=rep)

        out_pend[p] = pltpu.async_copy(tb, out_hbm.at[b, c0 + ci], sem_o)
    out_pend[0].wait()
    out_pend[1].wait()


def kernel(p, fc_pos_W, fc_pos_b, W0, b0, W1, b1, Ws, fc_c_W, fc_c_b):
    pt = jnp.transpose(p, (0, 2, 1))
    fwT = jnp.transpose(fc_pos_W)
    fbT = fc_pos_b[:, None]
    w0T = jnp.transpose(W0, (0, 2, 1))
    b0T = b0[:, :, None]
    w1T = jnp.transpose(W1, (0, 2, 1))
    b1T = b1[:, :, None]
    wsT = jnp.transpose(Ws, (0, 2, 1))
    fcwT = jnp.transpose(fc_c_W)
    fcbT = fc_c_b[:, None]

    idx, net = _prologue_call(pt, fwT, fbT, w0T[0], b0T[0], w1T[0], b1T[0],
                              wsT[0])
    for i in range(1, NB):
        pooled = _pool_call(idx, net)
        if i < NB - 1:
            net = _res_call(net, pooled, w0T[i], b0T[i], w1T[i], b1T[i],
                            wsT[i])
        else:
            c = _res_final_call(net, pooled, w0T[i], b0T[i], w1T[i], b1T[i],
                                wsT[i], fcwT, fcbT)
    plane = _mean_call(idx, c)
    return plane.reshape(B, C_DIM, RESO, RESO)


# speedup vs baseline: 2.0240x; 1.1425x over previous
import functools

import jax
import jax.numpy as jnp
from jax import lax
from jax.experimental import pallas as pl
from jax.experimental.pallas import tpu as pltpu
from jax.experimental.pallas import tpu_sc as plsc

B, T, DIM = 16, 4096, 3
HIDDEN = 32
H2 = 2 * HIDDEN
C_DIM = 32
RESO = 128
PAD = 0.1
NB = 5
S = RESO * RESO
L = 16
GROUPS = T // L
NEG = float(jnp.finfo(jnp.float32).min)


def _dot(a, b):
    return jax.lax.dot_general(a, b, (((1,), (0,)), ((), ())),
                               preferred_element_type=jnp.float32)


def _prologue_body(pt_ref, fw_ref, fb_ref, w0_ref, b0_ref, w1_ref, b1_ref,
                   ws_ref, idx_ref, net_ref):
    pt = pt_ref[0]
    x0 = pt[0:1, :]
    x1 = pt[2:3, :]

    def norm(v):
        vn = v / (1.0 + PAD + 10e-4)
        vn = vn + 0.5
        vn = jnp.where(vn >= 1.0, 1.0 - 10e-6, vn)
        vn = jnp.where(vn < 0.0, 0.0, vn)
        return vn

    xi0 = jnp.clip((norm(x0) * RESO).astype(jnp.int32), 0, RESO - 1)
    xi1 = jnp.clip((norm(x1) * RESO).astype(jnp.int32), 0, RESO - 1)
    idx_ref[0] = xi0 + RESO * xi1

    h = _dot(fw_ref[...], pt) + fb_ref[...]
    n0 = _dot(w0_ref[...], jnp.maximum(h, 0.0)) + b0_ref[...]
    dx = _dot(w1_ref[...], jnp.maximum(n0, 0.0)) + b1_ref[...]
    net_ref[0] = _dot(ws_ref[...], h) + dx


def _res_body(net_ref, pool_ref, w0_ref, b0_ref, w1_ref, b1_ref, ws_ref,
              out_ref):
    x = jnp.concatenate([net_ref[0], pool_ref[0]], axis=0)
    n0 = _dot(w0_ref[...], jnp.maximum(x, 0.0)) + b0_ref[...]
    dx = _dot(w1_ref[...], jnp.maximum(n0, 0.0)) + b1_ref[...]
    out_ref[0] = _dot(ws_ref[...], x) + dx


def _res_final_body(net_ref, pool_ref, w0_ref, b0_ref, w1_ref, b1_ref,
                    ws_ref, fcw_ref, fcb_ref, out_ref):
    x = jnp.concatenate([net_ref[0], pool_ref[0]], axis=0)
    n0 = _dot(w0_ref[...], jnp.maximum(x, 0.0)) + b0_ref[...]
    dx = _dot(w1_ref[...], jnp.maximum(n0, 0.0)) + b1_ref[...]
    net = _dot(ws_ref[...], x) + dx
    out_ref[0] = _dot(fcw_ref[...], net) + fcb_ref[...]


def _full(shape):
    return pl.BlockSpec(shape, lambda b: (0,) * len(shape))


def _row(shape):
    return pl.BlockSpec(shape, lambda b: (b,) + (0,) * (len(shape) - 1))


_prologue_call = pl.pallas_call(
    _prologue_body,
    grid=(B,),
    in_specs=[_row((1, DIM, T)), _full((H2, DIM)), _full((H2, 1)),
              _full((HIDDEN, H2)), _full((HIDDEN, 1)),
              _full((HIDDEN, HIDDEN)), _full((HIDDEN, 1)),
              _full((HIDDEN, H2))],
    out_specs=[_row((1, 1, T)), _row((1, HIDDEN, T))],
    out_shape=[jax.ShapeDtypeStruct((B, 1, T), jnp.int32),
               jax.ShapeDtypeStruct((B, HIDDEN, T), jnp.float32)],
)

_res_call = pl.pallas_call(
    _res_body,
    grid=(B,),
    in_specs=[_row((1, HIDDEN, T)), _row((1, HIDDEN, T)),
              _full((HIDDEN, H2)), _full((HIDDEN, 1)),
              _full((HIDDEN, HIDDEN)), _full((HIDDEN, 1)),
              _full((HIDDEN, H2))],
    out_specs=_row((1, HIDDEN, T)),
    out_shape=jax.ShapeDtypeStruct((B, HIDDEN, T), jnp.float32),
)

_res_final_call = pl.pallas_call(
    _res_final_body,
    grid=(B,),
    in_specs=[_row((1, HIDDEN, T)), _row((1, HIDDEN, T)),
              _full((HIDDEN, H2)), _full((HIDDEN, 1)),
              _full((HIDDEN, HIDDEN)), _full((HIDDEN, 1)),
              _full((HIDDEN, H2)), _full((C_DIM, HIDDEN)), _full((C_DIM, 1))],
    out_specs=_row((1, C_DIM, T)),
    out_shape=jax.ShapeDtypeStruct((B, C_DIM, T), jnp.float32),
)


_MESH = plsc.VectorSubcoreMesh(core_axis_name="c", subcore_axis_name="s")
_CH_PER_W = C_DIM // 2
_SC_PARAMS = pltpu.CompilerParams(needs_layout_passes=False)


def _build_schedule(idx_v, claim, rep_v, nf_pt_v, nf_rep_v):
    lanes = lax.iota(jnp.int32, L)

    @plsc.parallel_loop(0, GROUPS, unroll=4)
    def _(g):
        idxs = idx_v[pl.ds(g * L, L)]
        plsc.store_scatter(claim, [idxs], jnp.full((L,), -1, jnp.int32))

    def build_g(g, off):
        idxs = idx_v[pl.ds(g * L, L)]
        gids = g * L + lanes
        cur = plsc.load_gather(claim, [idxs])
        free = cur == -1
        plsc.store_scatter(claim, [idxs], gids, mask=free)
        got = plsc.load_gather(claim, [idxs])
        rep = free & (got == gids)
        rep_v[pl.ds(g * L, L)] = jnp.where(rep, 1, 0)
        nf = jnp.logical_not(rep)
        nf_i = jnp.where(nf, 1, 0)
        pos = off + plsc.cumsum(nf_i) - 1
        plsc.store_scatter(nf_pt_v, [pos], gids, mask=nf)
        repgid = jnp.where(free, got, cur)
        plsc.store_scatter(nf_rep_v, [pos], repgid, mask=nf)
        return off + jnp.sum(nf_i)

    return lax.fori_loop(0, GROUPS, build_g, jnp.int32(0))


@functools.partial(
    pl.kernel, mesh=_MESH,
    out_type=jax.ShapeDtypeStruct((B, C_DIM, T), jnp.float32),
    compiler_params=_SC_PARAMS,
    scratch_types=[pltpu.VMEM((T,), jnp.int32),
                   pltpu.VMEM((T,), jnp.float32),
                   pltpu.VMEM((T,), jnp.float32),
                   pltpu.VMEM((S,), jnp.float32),
                   pltpu.VMEM((T,), jnp.float32),
                   pltpu.VMEM((T,), jnp.float32),
                   pltpu.VMEM((S,), jnp.int32),
                   pltpu.VMEM((T,), jnp.int32),
                   pltpu.VMEM((T,), jnp.int32),
                   pltpu.VMEM((T,), jnp.int32),
                   pltpu.VMEM((L,), jnp.int32),
                   pltpu.VMEM((L,), jnp.float32),
                   pltpu.VMEM((T,), jnp.int32),
                   pltpu.VMEM((T,), jnp.int32),
                   pltpu.VMEM((T,), jnp.int32),
                   pltpu.VMEM((T,), jnp.int32),
                   pltpu.SemaphoreType.DMA,
                   pltpu.SemaphoreType.DMA,
                   pltpu.SemaphoreType.DMA,
                   pltpu.SemaphoreType.DMA],
)
def _pool_call(idx_hbm, net_hbm, out_hbm, idx_v, in_a, in_b, tab, out_a,
               out_b, claim, rep_v, nf_pt_v, nf_rep_v, shk, shv, nfs_pt,
               eqm_v, do_v, tgt_v, sem_ia, sem_ib, sem_oa, sem_ob):
    wid = lax.axis_index("c") * 16 + lax.axis_index("s")
    b = wid // 2
    c0 = (wid % 2) * _CH_PER_W
    pltpu.sync_copy(idx_hbm.at[b, 0], idx_v)
    in_pend = [pltpu.async_copy(net_hbm.at[b, c0], in_a, sem_ia),
               pltpu.async_copy(net_hbm.at[b, c0 + 1], in_b, sem_ib)]
    n_nf = _build_schedule(idx_v, claim, rep_v, nf_pt_v, nf_rep_v)
    n_nf_vregs = (n_nf + L - 1) // L
    lanes = lax.iota(jnp.int32, L)
    sent = jnp.int32(0x7FFFFFFF)

    def prep_k(k, carry):
        sl = pl.ds(k * L, L)
        valid = (k * L + lanes) < n_nf
        pts = jnp.where(valid, nf_pt_v[sl], 0)
        cells = plsc.load_gather(idx_v, [pts])
        cells = jnp.where(valid, cells, sent)
        kk, pv = plsc.sort_key_val(cells, pts)
        nfs_pt[sl] = pv
        shk[...] = kk
        eq = jnp.zeros((L,), jnp.int32)
        for i, s in enumerate((1, 2, 4, 8)):
            ks = plsc.load_gather(shk, [jnp.maximum(lanes - s, 0)])
            eq = eq | jnp.where((ks == kk) & (lanes >= s), 1 << i, 0)
        eqm_v[sl] = eq
        kn = plsc.load_gather(shk, [jnp.minimum(lanes + 1, L - 1)])
        do = ((kn != kk) | (lanes == L - 1)) & (kk != sent)
        do_v[sl] = jnp.where(do, 1, 0)
        tgt_v[sl] = jnp.where(do, kk, 0)
        return carry
    lax.fori_loop(0, n_nf_vregs, prep_k, 0)

    pend = [None, None]
    for ci in range(_CH_PER_W):
        p = ci % 2
        vv, ob = (in_a, out_a) if p == 0 else (in_b, out_b)
        sem_i, sem_o = (sem_ia, sem_oa) if p == 0 else (sem_ib, sem_ob)
        in_pend[p].wait()
        if pend[p] is not None:
            pend[p].wait()

        @plsc.parallel_loop(0, GROUPS, unroll=4)
        def _(g, vv=vv):
            sl = pl.ds(g * L, L)
            rep = rep_v[sl] != 0
            plsc.store_scatter(tab, [idx_v[sl]], vv[sl], mask=rep)

        def nf_k(k, carry2, vv=vv):
            sl = pl.ds(k * L, L)
            vx = plsc.load_gather(vv, [nfs_pt[sl]])
            eq = eqm_v[sl]
            for i, s in enumerate((1, 2, 4, 8)):
                shv[...] = vx
                vs = plsc.load_gather(shv, [jnp.maximum(lanes - s, 0)])
                vx = jnp.where((eq & (1 << i)) != 0, jnp.maximum(vx, vs), vx)
            do = do_v[sl] != 0
            tgt = tgt_v[sl]
            cur = plsc.load_gather(tab, [tgt])
            plsc.store_scatter(tab, [tgt], jnp.maximum(cur, vx), mask=do)
            return carry2
        lax.fori_loop(0, n_nf_vregs, nf_k, 0)

        @plsc.parallel_loop(0, GROUPS, unroll=4)
        def _(g, ob=ob):
            sl = pl.ds(g * L, L)
            ob[sl] = plsc.load_gather(tab, [idx_v[sl]])

        if ci + 2 < _CH_PER_W:
            in_pend[p] = pltpu.async_copy(net_hbm.at[b, c0 + ci + 2], vv,
                                          sem_i)
        pend[p] = pltpu.async_copy(ob, out_hbm.at[b, c0 + ci], sem_o)
    pend[0].wait()
    pend[1].wait()


@functools.partial(
    pl.kernel, mesh=_MESH,
    out_type=jax.ShapeDtypeStruct((B, C_DIM, S), jnp.float32),
    compiler_params=_SC_PARAMS,
    scratch_types=[pltpu.VMEM((T,), jnp.int32),
                   pltpu.VMEM((T,), jnp.float32),
                   pltpu.VMEM((T,), jnp.float32),
                   pltpu.VMEM((T,), jnp.float32),
                   pltpu.VMEM((T,), jnp.float32),
                   pltpu.VMEM((T,), jnp.float32),
                   pltpu.VMEM((S,), jnp.float32),
                   pltpu.VMEM((S,), jnp.float32),
                   pltpu.VMEM((S,), jnp.int32),
                   pltpu.VMEM((T,), jnp.int32),
                   pltpu.VMEM((T,), jnp.int32),
                   pltpu.VMEM((T,), jnp.int32),
                   pltpu.VMEM((T,), jnp.float32),
                   pltpu.VMEM((T,), jnp.int32),
                   pltpu.VMEM((L,), jnp.int32),
                   pltpu.VMEM((L,), jnp.float32),
                   pltpu.SemaphoreType.DMA,
                   pltpu.SemaphoreType.DMA,
                   pltpu.SemaphoreType.DMA,
                   pltpu.SemaphoreType.DMA],
)
def _mean_call(idx_hbm, c_hbm, out_hbm, idx_v, in_a, in_b, sv_a, sv_b,
               rec_pt, tab_a, tab_b, claim, rep_v, nf_pt_v, nf_rep_v,
               cnts, repof_v, shk, shv, sem_ia, sem_ib, sem_oa, sem_ob):
    wid = lax.axis_index("c") * 16 + lax.axis_index("s")
    b = wid // 2
    c0 = (wid % 2) * _CH_PER_W
    pltpu.sync_copy(idx_hbm.at[b, 0], idx_v)
    in_pend = [pltpu.async_copy(c_hbm.at[b, c0], in_a, sem_ia),
               pltpu.async_copy(c_hbm.at[b, c0 + 1], in_b, sem_ib)]
    n_nf = _build_schedule(idx_v, claim, rep_v, nf_pt_v, nf_rep_v)
    n_nf_vregs = (n_nf + L - 1) // L
    lanes = lax.iota(jnp.int32, L)

    @plsc.parallel_loop(0, GROUPS, unroll=4)
    def _(g):
        sl = pl.ds(g * L, L)
        cells = idx_v[sl]
        cnts[sl] = jnp.ones((L,), jnp.float32)
        repof_v[sl] = plsc.load_gather(claim, [cells])

    @plsc.parallel_loop(0, S // L, unroll=4)
    def _(g):
        tab_a[pl.ds(g * L, L)] = jnp.zeros((L,), jnp.float32)

    @plsc.parallel_loop(0, S // L, unroll=4)
    def _(g):
        tab_b[pl.ds(g * L, L)] = jnp.zeros((L,), jnp.float32)

    zero_mask = jnp.zeros((L,), jnp.bool_)

    _SENT = jnp.int32(0x7FFFFFFF)

    def _fold(dst, vals_fn):
        def fold_k(k, carry):
            sl = pl.ds(k * L, L)
            valid = (k * L + lanes) < n_nf
            pts = jnp.where(valid, nf_pt_v[sl], 0)
            reps = jnp.where(valid, nf_rep_v[sl], _SENT)
            vals = jnp.where(valid, vals_fn(pts), 0.0)
            kk, vv = plsc.sort_key_val(reps, vals)
            shk[...] = kk
            for s in (1, 2, 4, 8):
                idxs = jnp.maximum(lanes - s, 0)
                ks = plsc.load_gather(shk, [idxs])
                shv[...] = vv
                vs = plsc.load_gather(shv, [idxs])
                vv = jnp.where((ks == kk) & (lanes >= s), vv + vs, vv)
            kn = plsc.load_gather(shk, [jnp.minimum(lanes + 1, L - 1)])
            do = ((kn != kk) | (lanes == L - 1)) & (kk != _SENT)
            tgt = jnp.where(do, kk, 0)
            cur = plsc.load_gather(dst, [tgt])
            plsc.store_scatter(dst, [tgt], cur + vv, mask=do)
            return carry
        lax.fori_loop(0, n_nf_vregs, fold_k, 0)

    _fold(cnts, lambda pts: jnp.ones((L,), jnp.float32))

    @plsc.parallel_loop(0, GROUPS, unroll=4)
    def _(g):
        sl = pl.ds(g * L, L)
        c = plsc.load_gather(cnts, [repof_v[sl]])
        rec_pt[sl] = 1.0 / c

    out_pend = [None, None]
    for ci in range(_CH_PER_W):
        p = ci % 2
        vv, sv, tb = (in_a, sv_a, tab_a) if p == 0 else (in_b, sv_b, tab_b)
        sem_i, sem_o = (sem_ia, sem_oa) if p == 0 else (sem_ib, sem_ob)
        in_pend[p].wait()

        @plsc.parallel_loop(0, GROUPS, unroll=4)
        def _(g, vv=vv, sv=sv):
            sl = pl.ds(g * L, L)
            sv[sl] = vv[sl] * rec_pt[sl]

        if ci + 2 < _CH_PER_W:
            in_pend[p] = pltpu.async_copy(c_hbm.at[b, c0 + ci + 2], vv, sem_i)

        def _sv_vals(pts, sv=sv):
            return plsc.load_gather(sv, [pts])
        _fold(sv, _sv_vals)

        if out_pend[p] is not None:
            out_pend[p].wait()

        @plsc.parallel_loop(0, GROUPS, unroll=4)
        def _(g, sv=sv, tb=tb):
            sl = pl.ds(g * L, L)
            rep = rep_v[sl] != 0
            plsc.store_scatter(tb, [idx_v[sl]], sv[sl], mask=rep)

        out_pend[p] = pltpu.async_copy(tb, out_hbm.at[b, c0 + ci], sem_o)
    out_pend[0].wait()
    out_pend[1].wait()


def kernel(p, fc_pos_W, fc_pos_b, W0, b0, W1, b1, Ws, fc_c_W, fc_c_b):
    pt = jnp.transpose(p, (0, 2, 1))
    fwT = jnp.transpose(fc_pos_W)
    fbT = fc_pos_b[:, None]
    w0T = jnp.transpose(W0, (0, 2, 1))
    b0T = b0[:, :, None]
    w1T = jnp.transpose(W1, (0, 2, 1))
    b1T = b1[:, :, None]
    wsT = jnp.transpose(Ws, (0, 2, 1))
    fcwT = jnp.transpose(fc_c_W)
    fcbT = fc_c_b[:, None]

    idx, net = _prologue_call(pt, fwT, fbT, w0T[0], b0T[0], w1T[0], b1T[0],
                              wsT[0])
    for i in range(1, NB):
        pooled = _pool_call(idx, net)
        if i < NB - 1:
            net = _res_call(net, pooled, w0T[i], b0T[i], w1T[i], b1T[i],
                            wsT[i])
        else:
            c = _res_final_call(net, pooled, w0T[i], b0T[i], w1T[i], b1T[i],
                                wsT[i], fcwT, fcbT)
    plane = _mean_call(idx, c)
    return plane.reshape(B, C_DIM, RESO, RESO)


# speedup vs baseline: 2.1134x; 1.0442x over previous
import functools

import jax
import jax.numpy as jnp
from jax import lax
from jax.experimental import pallas as pl
from jax.experimental.pallas import tpu as pltpu
from jax.experimental.pallas import tpu_sc as plsc

B, T, DIM = 16, 4096, 3
HIDDEN = 32
H2 = 2 * HIDDEN
C_DIM = 32
RESO = 128
PAD = 0.1
NB = 5
S = RESO * RESO
L = 16
GROUPS = T // L
NEG = float(jnp.finfo(jnp.float32).min)


def _dot(a, b):
    return jax.lax.dot_general(a, b, (((1,), (0,)), ((), ())),
                               preferred_element_type=jnp.float32)


def _prologue_body(pt_ref, fw_ref, fb_ref, w0_ref, b0_ref, w1_ref, b1_ref,
                   ws_ref, idx_ref, net_ref):
    pt = pt_ref[0]
    x0 = pt[0:1, :]
    x1 = pt[2:3, :]

    def norm(v):
        vn = v / (1.0 + PAD + 10e-4)
        vn = vn + 0.5
        vn = jnp.where(vn >= 1.0, 1.0 - 10e-6, vn)
        vn = jnp.where(vn < 0.0, 0.0, vn)
        return vn

    xi0 = jnp.clip((norm(x0) * RESO).astype(jnp.int32), 0, RESO - 1)
    xi1 = jnp.clip((norm(x1) * RESO).astype(jnp.int32), 0, RESO - 1)
    idx_ref[0] = xi0 + RESO * xi1

    h = _dot(fw_ref[...], pt) + fb_ref[...]
    n0 = _dot(w0_ref[...], jnp.maximum(h, 0.0)) + b0_ref[...]
    dx = _dot(w1_ref[...], jnp.maximum(n0, 0.0)) + b1_ref[...]
    net_ref[0] = _dot(ws_ref[...], h) + dx


def _res_body(net_ref, pool_ref, w0_ref, b0_ref, w1_ref, b1_ref, ws_ref,
              out_ref):
    x = jnp.concatenate([net_ref[0], pool_ref[0]], axis=0)
    n0 = _dot(w0_ref[...], jnp.maximum(x, 0.0)) + b0_ref[...]
    dx = _dot(w1_ref[...], jnp.maximum(n0, 0.0)) + b1_ref[...]
    out_ref[0] = _dot(ws_ref[...], x) + dx


def _res_final_body(net_ref, pool_ref, w0_ref, b0_ref, w1_ref, b1_ref,
                    ws_ref, fcw_ref, fcb_ref, out_ref):
    x = jnp.concatenate([net_ref[0], pool_ref[0]], axis=0)
    n0 = _dot(w0_ref[...], jnp.maximum(x, 0.0)) + b0_ref[...]
    dx = _dot(w1_ref[...], jnp.maximum(n0, 0.0)) + b1_ref[...]
    net = _dot(ws_ref[...], x) + dx
    out_ref[0] = _dot(fcw_ref[...], net) + fcb_ref[...]


def _full(shape):
    return pl.BlockSpec(shape, lambda b: (0,) * len(shape))


def _row(shape):
    return pl.BlockSpec(shape, lambda b: (b,) + (0,) * (len(shape) - 1))


_prologue_call = pl.pallas_call(
    _prologue_body,
    grid=(B,),
    in_specs=[_row((1, DIM, T)), _full((H2, DIM)), _full((H2, 1)),
              _full((HIDDEN, H2)), _full((HIDDEN, 1)),
              _full((HIDDEN, HIDDEN)), _full((HIDDEN, 1)),
              _full((HIDDEN, H2))],
    out_specs=[_row((1, 1, T)), _row((1, HIDDEN, T))],
    out_shape=[jax.ShapeDtypeStruct((B, 1, T), jnp.int32),
               jax.ShapeDtypeStruct((B, HIDDEN, T), jnp.float32)],
)

_res_call = pl.pallas_call(
    _res_body,
    grid=(B,),
    in_specs=[_row((1, HIDDEN, T)), _row((1, HIDDEN, T)),
              _full((HIDDEN, H2)), _full((HIDDEN, 1)),
              _full((HIDDEN, HIDDEN)), _full((HIDDEN, 1)),
              _full((HIDDEN, H2))],
    out_specs=_row((1, HIDDEN, T)),
    out_shape=jax.ShapeDtypeStruct((B, HIDDEN, T), jnp.float32),
)

_res_final_call = pl.pallas_call(
    _res_final_body,
    grid=(B,),
    in_specs=[_row((1, HIDDEN, T)), _row((1, HIDDEN, T)),
              _full((HIDDEN, H2)), _full((HIDDEN, 1)),
              _full((HIDDEN, HIDDEN)), _full((HIDDEN, 1)),
              _full((HIDDEN, H2)), _full((C_DIM, HIDDEN)), _full((C_DIM, 1))],
    out_specs=_row((1, C_DIM, T)),
    out_shape=jax.ShapeDtypeStruct((B, C_DIM, T), jnp.float32),
)


_MESH = plsc.VectorSubcoreMesh(core_axis_name="c", subcore_axis_name="s")
_CH_PER_W = C_DIM // 2
_SC_PARAMS = pltpu.CompilerParams(needs_layout_passes=False)


def _build_schedule(idx_v, claim, rep_v, nf_pt_v, nf_rep_v):
    lanes = lax.iota(jnp.int32, L)

    @plsc.parallel_loop(0, GROUPS, unroll=4)
    def _(g):
        idxs = idx_v[pl.ds(g * L, L)]
        plsc.store_scatter(claim, [idxs], jnp.full((L,), -1, jnp.int32))

    def build_g(g, off):
        idxs = idx_v[pl.ds(g * L, L)]
        gids = g * L + lanes
        cur = plsc.load_gather(claim, [idxs])
        free = cur == -1
        plsc.store_scatter(claim, [idxs], gids, mask=free)
        got = plsc.load_gather(claim, [idxs])
        rep = free & (got == gids)
        rep_v[pl.ds(g * L, L)] = jnp.where(rep, 1, 0)
        nf = jnp.logical_not(rep)
        nf_i = jnp.where(nf, 1, 0)
        pos = off + plsc.cumsum(nf_i) - 1
        plsc.store_scatter(nf_pt_v, [pos], gids, mask=nf)
        repgid = jnp.where(free, got, cur)
        plsc.store_scatter(nf_rep_v, [pos], repgid, mask=nf)
        return off + jnp.sum(nf_i)

    return lax.fori_loop(0, GROUPS, build_g, jnp.int32(0))


@functools.partial(
    pl.kernel, mesh=_MESH,
    out_type=jax.ShapeDtypeStruct((B, C_DIM, T), jnp.float32),
    compiler_params=_SC_PARAMS,
    scratch_types=[pltpu.VMEM((T,), jnp.int32),
                   pltpu.VMEM((T,), jnp.float32),
                   pltpu.VMEM((T,), jnp.float32),
                   pltpu.VMEM((S,), jnp.float32),
                   pltpu.VMEM((T,), jnp.float32),
                   pltpu.VMEM((T,), jnp.float32),
                   pltpu.VMEM((S,), jnp.int32),
                   pltpu.VMEM((T,), jnp.int32),
                   pltpu.VMEM((T,), jnp.int32),
                   pltpu.VMEM((T,), jnp.int32),
                   pltpu.VMEM((L,), jnp.int32),
                   pltpu.VMEM((L,), jnp.float32),
                   pltpu.VMEM((T,), jnp.int32),
                   pltpu.VMEM((T,), jnp.int32),
                   pltpu.VMEM((T,), jnp.int32),
                   pltpu.VMEM((T,), jnp.int32),
                   pltpu.SemaphoreType.DMA,
                   pltpu.SemaphoreType.DMA,
                   pltpu.SemaphoreType.DMA,
                   pltpu.SemaphoreType.DMA],
)
def _pool_call(idx_hbm, net_hbm, out_hbm, idx_v, in_a, in_b, tab, out_a,
               out_b, claim, rep_v, nf_pt_v, nf_rep_v, shk, shv, nfs_pt,
               eqm_v, do_v, tgt_v, sem_ia, sem_ib, sem_oa, sem_ob):
    wid = lax.axis_index("c") * 16 + lax.axis_index("s")
    b = wid // 2
    c0 = (wid % 2) * _CH_PER_W
    pltpu.sync_copy(idx_hbm.at[b, 0], idx_v)
    in_pend = [pltpu.async_copy(net_hbm.at[b, c0], in_a, sem_ia),
               pltpu.async_copy(net_hbm.at[b, c0 + 1], in_b, sem_ib)]
    n_nf = _build_schedule(idx_v, claim, rep_v, nf_pt_v, nf_rep_v)
    n_nf_vregs = (n_nf + L - 1) // L
    lanes = lax.iota(jnp.int32, L)
    sent = jnp.int32(0x7FFFFFFF)

    def prep_k(k, carry):
        sl = pl.ds(k * L, L)
        valid = (k * L + lanes) < n_nf
        pts = jnp.where(valid, nf_pt_v[sl], 0)
        cells = plsc.load_gather(idx_v, [pts])
        cells = jnp.where(valid, cells, sent)
        kk, pv = plsc.sort_key_val(cells, pts)
        nfs_pt[sl] = pv
        shk[...] = kk
        eq = jnp.zeros((L,), jnp.int32)
        for i, s in enumerate((1, 2, 4, 8)):
            ks = plsc.load_gather(shk, [jnp.maximum(lanes - s, 0)])
            eq = eq | jnp.where((ks == kk) & (lanes >= s), 1 << i, 0)
        eqm_v[sl] = eq
        kn = plsc.load_gather(shk, [jnp.minimum(lanes + 1, L - 1)])
        do = ((kn != kk) | (lanes == L - 1)) & (kk != sent)
        do_v[sl] = jnp.where(do, 1, 0)
        tgt_v[sl] = jnp.where(do, kk, 0)
        return carry
    lax.fori_loop(0, n_nf_vregs, prep_k, 0)

    pend = [None, None]
    for ci in range(_CH_PER_W):
        p = ci % 2
        vv, ob = (in_a, out_a) if p == 0 else (in_b, out_b)
        sem_i, sem_o = (sem_ia, sem_oa) if p == 0 else (sem_ib, sem_ob)
        in_pend[p].wait()
        if pend[p] is not None:
            pend[p].wait()

        @plsc.parallel_loop(0, GROUPS, unroll=4)
        def _(g, vv=vv):
            sl = pl.ds(g * L, L)
            rep = rep_v[sl] != 0
            plsc.store_scatter(tab, [idx_v[sl]], vv[sl], mask=rep)

        def nf_k(k, carry2, vv=vv):
            sl = pl.ds(k * L, L)
            vx = plsc.load_gather(vv, [nfs_pt[sl]])
            eq = eqm_v[sl]
            for i, s in enumerate((1, 2, 4, 8)):
                shv[...] = vx
                vs = plsc.load_gather(shv, [jnp.maximum(lanes - s, 0)])
                vx = jnp.where((eq & (1 << i)) != 0, jnp.maximum(vx, vs), vx)
            do = do_v[sl] != 0
            tgt = tgt_v[sl]
            cur = plsc.load_gather(tab, [tgt])
            plsc.store_scatter(tab, [tgt], jnp.maximum(cur, vx), mask=do)
            return carry2
        lax.fori_loop(0, n_nf_vregs, nf_k, 0)

        @plsc.parallel_loop(0, GROUPS, unroll=4)
        def _(g, ob=ob):
            sl = pl.ds(g * L, L)
            ob[sl] = plsc.load_gather(tab, [idx_v[sl]])

        if ci + 2 < _CH_PER_W:
            in_pend[p] = pltpu.async_copy(net_hbm.at[b, c0 + ci + 2], vv,
                                          sem_i)
        pend[p] = pltpu.async_copy(ob, out_hbm.at[b, c0 + ci], sem_o)
    pend[0].wait()
    pend[1].wait()


@functools.partial(
    pl.kernel, mesh=_MESH,
    out_type=jax.ShapeDtypeStruct((B, C_DIM, S), jnp.float32),
    compiler_params=_SC_PARAMS,
    scratch_types=[pltpu.VMEM((T,), jnp.int32),
                   pltpu.VMEM((T,), jnp.float32),
                   pltpu.VMEM((T,), jnp.float32),
                   pltpu.VMEM((T,), jnp.float32),
                   pltpu.VMEM((T,), jnp.float32),
                   pltpu.VMEM((T,), jnp.float32),
                   pltpu.VMEM((S,), jnp.float32),
                   pltpu.VMEM((S,), jnp.float32),
                   pltpu.VMEM((S,), jnp.int32),
                   pltpu.VMEM((T,), jnp.int32),
                   pltpu.VMEM((T,), jnp.int32),
                   pltpu.VMEM((T,), jnp.int32),
                   pltpu.VMEM((T,), jnp.float32),
                   pltpu.VMEM((T,), jnp.int32),
                   pltpu.VMEM((L,), jnp.int32),
                   pltpu.VMEM((L,), jnp.float32),
                   pltpu.VMEM((T,), jnp.int32),
                   pltpu.VMEM((T,), jnp.int32),
                   pltpu.VMEM((T,), jnp.int32),
                   pltpu.VMEM((T,), jnp.int32),
                   pltpu.SemaphoreType.DMA,
                   pltpu.SemaphoreType.DMA,
                   pltpu.SemaphoreType.DMA,
                   pltpu.SemaphoreType.DMA],
)
def _mean_call(idx_hbm, c_hbm, out_hbm, idx_v, in_a, in_b, sv_a, sv_b,
               rec_pt, tab_a, tab_b, claim, rep_v, nf_pt_v, nf_rep_v,
               cnts, repof_v, shk, shv, nfs_pt, eqm_v, do_v, tgt_v,
               sem_ia, sem_ib, sem_oa, sem_ob):
    wid = lax.axis_index("c") * 16 + lax.axis_index("s")
    b = wid // 2
    c0 = (wid % 2) * _CH_PER_W
    pltpu.sync_copy(idx_hbm.at[b, 0], idx_v)
    in_pend = [pltpu.async_copy(c_hbm.at[b, c0], in_a, sem_ia),
               pltpu.async_copy(c_hbm.at[b, c0 + 1], in_b, sem_ib)]
    n_nf = _build_schedule(idx_v, claim, rep_v, nf_pt_v, nf_rep_v)
    n_nf_vregs = (n_nf + L - 1) // L
    lanes = lax.iota(jnp.int32, L)

    @plsc.parallel_loop(0, GROUPS, unroll=4)
    def _(g):
        sl = pl.ds(g * L, L)
        cells = idx_v[sl]
        cnts[sl] = jnp.ones((L,), jnp.float32)
        repof_v[sl] = plsc.load_gather(claim, [cells])

    @plsc.parallel_loop(0, S // L, unroll=4)
    def _(g):
        tab_a[pl.ds(g * L, L)] = jnp.zeros((L,), jnp.float32)

    @plsc.parallel_loop(0, S // L, unroll=4)
    def _(g):
        tab_b[pl.ds(g * L, L)] = jnp.zeros((L,), jnp.float32)

    zero_mask = jnp.zeros((L,), jnp.bool_)

    _SENT = jnp.int32(0x7FFFFFFF)

    def prep_k(k, carry):
        sl = pl.ds(k * L, L)
        valid = (k * L + lanes) < n_nf
        pts = jnp.where(valid, nf_pt_v[sl], 0)
        reps = jnp.where(valid, nf_rep_v[sl], _SENT)
        kk, pv = plsc.sort_key_val(reps, pts)
        nfs_pt[sl] = pv
        shk[...] = kk
        eq = jnp.zeros((L,), jnp.int32)
        for i, s in enumerate((1, 2, 4, 8)):
            ks = plsc.load_gather(shk, [jnp.maximum(lanes - s, 0)])
            eq = eq | jnp.where((ks == kk) & (lanes >= s), 1 << i, 0)
        eqm_v[sl] = eq
        kn = plsc.load_gather(shk, [jnp.minimum(lanes + 1, L - 1)])
        do = ((kn != kk) | (lanes == L - 1)) & (kk != _SENT)
        do_v[sl] = jnp.where(do, 1, 0)
        tgt_v[sl] = jnp.where(do, kk, 0)
        return carry
    lax.fori_loop(0, n_nf_vregs, prep_k, 0)

    def _fold(dst, vals_fn):
        def fold_k(k, carry):
            sl = pl.ds(k * L, L)
            vv = vals_fn(nfs_pt[sl])
            eq = eqm_v[sl]
            for i, s in enumerate((1, 2, 4, 8)):
                shv[...] = vv
                vs = plsc.load_gather(shv, [jnp.maximum(lanes - s, 0)])
                vv = jnp.where((eq & (1 << i)) != 0, vv + vs, vv)
            do = do_v[sl] != 0
            tgt = tgt_v[sl]
            cur = plsc.load_gather(dst, [tgt])
            plsc.store_scatter(dst, [tgt], cur + vv, mask=do)
            return carry
        lax.fori_loop(0, n_nf_vregs, fold_k, 0)

    _fold(cnts, lambda pts: jnp.ones((L,), jnp.float32))

    @plsc.parallel_loop(0, GROUPS, unroll=4)
    def _(g):
        sl = pl.ds(g * L, L)
        c = plsc.load_gather(cnts, [repof_v[sl]])
        rec_pt[sl] = 1.0 / c

    out_pend = [None, None]
    for ci in range(_CH_PER_W):
        p = ci % 2
        vv, sv, tb = (in_a, sv_a, tab_a) if p == 0 else (in_b, sv_b, tab_b)
        sem_i, sem_o = (sem_ia, sem_oa) if p == 0 else (sem_ib, sem_ob)
        in_pend[p].wait()

        @plsc.parallel_loop(0, GROUPS, unroll=4)
        def _(g, vv=vv, sv=sv):
            sl = pl.ds(g * L, L)
            sv[sl] = vv[sl] * rec_pt[sl]

        if ci + 2 < _CH_PER_W:
            in_pend[p] = pltpu.async_copy(c_hbm.at[b, c0 + ci + 2], vv, sem_i)

        def _sv_vals(pts, sv=sv):
            return plsc.load_gather(sv, [pts])
        _fold(sv, _sv_vals)

        if out_pend[p] is not None:
            out_pend[p].wait()

        @plsc.parallel_loop(0, GROUPS, unroll=4)
        def _(g, sv=sv, tb=tb):
            sl = pl.ds(g * L, L)
            rep = rep_v[sl] != 0
            plsc.store_scatter(tb, [idx_v[sl]], sv[sl], mask=rep)

        out_pend[p] = pltpu.async_copy(tb, out_hbm.at[b, c0 + ci], sem_o)
    out_pend[0].wait()
    out_pend[1].wait()


def kernel(p, fc_pos_W, fc_pos_b, W0, b0, W1, b1, Ws, fc_c_W, fc_c_b):
    pt = jnp.transpose(p, (0, 2, 1))
    fwT = jnp.transpose(fc_pos_W)
    fbT = fc_pos_b[:, None]
    w0T = jnp.transpose(W0, (0, 2, 1))
    b0T = b0[:, :, None]
    w1T = jnp.transpose(W1, (0, 2, 1))
    b1T = b1[:, :, None]
    wsT = jnp.transpose(Ws, (0, 2, 1))
    fcwT = jnp.transpose(fc_c_W)
    fcbT = fc_c_b[:, None]

    idx, net = _prologue_call(pt, fwT, fbT, w0T[0], b0T[0], w1T[0], b1T[0],
                              wsT[0])
    for i in range(1, NB):
        pooled = _pool_call(idx, net)
        if i < NB - 1:
            net = _res_call(net, pooled, w0T[i], b0T[i], w1T[i], b1T[i],
                            wsT[i])
        else:
            c = _res_final_call(net, pooled, w0T[i], b0T[i], w1T[i], b1T[i],
                                wsT[i], fcwT, fcbT)
    plane = _mean_call(idx, c)
    return plane.reshape(B, C_DIM, RESO, RESO)
